# Initial kernel scaffold; baseline (speedup 1.0000x reference)
import jax
import jax.numpy as jnp
from jax import lax
from jax.experimental import pallas as pl
from jax.experimental.pallas import tpu as pltpu
from jax.experimental.pallas import tpu_sc as plsc

N = 10000
E = 320000
NF = 9
VOCAB = 119
EMB = 128
HID = 256
NG = 64

NC = 2
NS = 16
NW = NC * NS

_mesh = plsc.VectorSubcoreMesh(core_axis_name="c", subcore_axis_name="s")


KN = 80
NCHUNK = N // KN
KD = 80
EPT_DEG = E // NW


def _sc_embed_deg_body(xT_hbm, emb_hbm, dst_hbm, h0_hbm, degp_hbm,
                       idxb, gbuf, abuf, oneb, dstb, zb, deg_sh):
    c = lax.axis_index("c")
    s = lax.axis_index("s")
    w = c * NS + s
    for r in range(40):
        zb[pl.ds(r * 16, 16)] = jnp.zeros((16,), jnp.float32)
    r0 = s * 640

    @pl.when(s < 15)
    def _():
        pltpu.sync_copy(zb, deg_sh.at[pl.ds(r0, 640)])

    @pl.when(s == 15)
    def _():
        pltpu.sync_copy(zb.at[pl.ds(0, 400)], deg_sh.at[pl.ds(9600, 400)])

    plsc.subcore_barrier()

    for r in range(5):
        oneb[pl.ds(r * 16, 16)] = jnp.ones((16,), jnp.float32)

    ebase = w * EPT_DEG

    def deg_chunk(k, carry):
        off = ebase + k * KD
        pltpu.sync_copy(dst_hbm.at[pl.ds(off, KD)], dstb)
        pltpu.sync_copy(oneb, deg_sh.at[dstb], add=True)
        return carry

    lax.fori_loop(0, EPT_DEG // KD, deg_chunk, 0)

    for i in range((NCHUNK + NW - 1) // NW):
        cid = w + i * NW

        @pl.when(cid < NCHUNK)
        def _():
            nb = cid * KN
            for f in range(NF):
                pltpu.sync_copy(xT_hbm.at[f, pl.ds(nb, KN)], idxb)
                if f == 0:
                    pltpu.sync_copy(emb_hbm.at[idxb], abuf)
                else:
                    pltpu.sync_copy(emb_hbm.at[idxb], gbuf)

                    def addrow(r, carry):
                        for cc in range(EMB // 16):
                            plsc.addupdate(abuf.at[r, pl.ds(cc * 16, 16)],
                                           gbuf[r, pl.ds(cc * 16, 16)])
                        return carry

                    lax.fori_loop(0, KN, addrow, 0)
            pltpu.sync_copy(abuf, h0_hbm.at[pl.ds(nb, KN)])

    plsc.subcore_barrier()

    @pl.when(s < 15)
    def _():
        pltpu.sync_copy(deg_sh.at[pl.ds(r0, 640)], zb)
        pltpu.sync_copy(zb, degp_hbm.at[c, pl.ds(r0, 640)])

    @pl.when(s == 15)
    def _():
        pltpu.sync_copy(deg_sh.at[pl.ds(9600, 400)], zb.at[pl.ds(0, 400)])
        pltpu.sync_copy(zb.at[pl.ds(0, 400)], degp_hbm.at[c, pl.ds(9600, 400)])


_sc_embed_deg = pl.kernel(
    _sc_embed_deg_body,
    out_type=[jax.ShapeDtypeStruct((N, EMB), jnp.float32),
              jax.ShapeDtypeStruct((2, N), jnp.float32)],
    mesh=_mesh,
    scratch_types=[pltpu.VMEM((KN,), jnp.int32),
                   pltpu.VMEM((KN, EMB), jnp.float32),
                   pltpu.VMEM((KN, EMB), jnp.float32),
                   pltpu.VMEM((KD,), jnp.float32),
                   pltpu.VMEM((KD,), jnp.int32),
                   pltpu.VMEM((640,), jnp.float32),
                   pltpu.VMEM_SHARED((N,), jnp.float32)],
)


KE = 128
EPT = E // NS
NFULL = EPT // KE
KR = EPT - NFULL * KE


def _make_edge_body(W):
    def body(src_hbm, dst_hbm, qlo_hbm, qhi_hbm, alo_hbm, ahi_hbm,
             sb, db, sbr, dbr, gbuf, zb2, acc_sh):
        c = lax.axis_index("c")
        s = lax.axis_index("s")

        def zrow(r, carry):
            for cc in range(W // 16):
                zb2[r, pl.ds(cc * 16, 16)] = jnp.zeros((16,), jnp.float32)
            return carry

        lax.fori_loop(0, 640, zrow, 0)
        r0 = s * 640

        @pl.when(s < 15)
        def _():
            pltpu.sync_copy(zb2, acc_sh.at[pl.ds(r0, 640)])

        @pl.when(s == 15)
        def _():
            pltpu.sync_copy(zb2.at[pl.ds(0, 400)], acc_sh.at[pl.ds(9600, 400)])

        plsc.subcore_barrier()
        ebase = s * EPT

        def chunk(k, carry):
            off = ebase + k * KE
            pltpu.sync_copy(src_hbm.at[pl.ds(off, KE)], sb)
            pltpu.sync_copy(dst_hbm.at[pl.ds(off, KE)], db)

            @pl.when(c == 0)
            def _():
                pltpu.sync_copy(qlo_hbm.at[sb], gbuf)

            @pl.when(c == 1)
            def _():
                pltpu.sync_copy(qhi_hbm.at[sb], gbuf)

            pltpu.sync_copy(gbuf, acc_sh.at[db], add=True)
            return carry

        lax.fori_loop(0, NFULL, chunk, 0)

        offr = ebase + NFULL * KE
        pltpu.sync_copy(src_hbm.at[pl.ds(offr, KR)], sbr)
        pltpu.sync_copy(dst_hbm.at[pl.ds(offr, KR)], dbr)

        @pl.when(c == 0)
        def _():
            pltpu.sync_copy(qlo_hbm.at[sbr], gbuf.at[pl.ds(0, KR)])

        @pl.when(c == 1)
        def _():
            pltpu.sync_copy(qhi_hbm.at[sbr], gbuf.at[pl.ds(0, KR)])

        pltpu.sync_copy(gbuf.at[pl.ds(0, KR)], acc_sh.at[dbr], add=True)
        plsc.subcore_barrier()

        @pl.when(s < 15)
        def _():
            pltpu.sync_copy(acc_sh.at[pl.ds(r0, 640)], zb2)

            @pl.when(c == 0)
            def _():
                pltpu.sync_copy(zb2, alo_hbm.at[pl.ds(r0, 640)])

            @pl.when(c == 1)
            def _():
                pltpu.sync_copy(zb2, ahi_hbm.at[pl.ds(r0, 640)])

        @pl.when(s == 15)
        def _():
            pltpu.sync_copy(acc_sh.at[pl.ds(9600, 400)], zb2.at[pl.ds(0, 400)])

            @pl.when(c == 0)
            def _():
                pltpu.sync_copy(zb2.at[pl.ds(0, 400)],
                                alo_hbm.at[pl.ds(9600, 400)])

            @pl.when(c == 1)
            def _():
                pltpu.sync_copy(zb2.at[pl.ds(0, 400)],
                                ahi_hbm.at[pl.ds(9600, 400)])

    return body


def _make_edge_agg(W):
    return pl.kernel(
        _make_edge_body(W),
        out_type=[jax.ShapeDtypeStruct((N, W), jnp.float32),
                  jax.ShapeDtypeStruct((N, W), jnp.float32)],
        mesh=_mesh,
        scratch_types=[pltpu.VMEM((KE,), jnp.int32),
                       pltpu.VMEM((KE,), jnp.int32),
                       pltpu.VMEM((KR,), jnp.int32),
                       pltpu.VMEM((KR,), jnp.int32),
                       pltpu.VMEM((KE, W), jnp.float32),
                       pltpu.VMEM((640, W), jnp.float32),
                       pltpu.VMEM_SHARED((N, W), jnp.float32)],
    )


_edge_agg_64 = _make_edge_agg(64)
_edge_agg_128 = _make_edge_agg(128)


BLK = 1000
NB = N // BLK


def _prescale_body(h0_ref, dpT_ref, s_ref, qlo_ref, qhi_ref):
    d = dpT_ref[:, 0:1] + dpT_ref[:, 1:2]
    s = lax.rsqrt(1.0 + d)
    q = h0_ref[...] * s
    s_ref[...] = s
    qlo_ref[...] = q[:, :64]
    qhi_ref[...] = q[:, 64:]


def _tc_prescale(h0, dpT):
    return pl.pallas_call(
        _prescale_body,
        grid=(NB,),
        in_specs=[pl.BlockSpec((BLK, EMB), lambda i: (i, 0)),
                  pl.BlockSpec((BLK, 2), lambda i: (i, 0))],
        out_specs=[pl.BlockSpec((BLK, 1), lambda i: (i, 0)),
                   pl.BlockSpec((BLK, 64), lambda i: (i, 0)),
                   pl.BlockSpec((BLK, 64), lambda i: (i, 0))],
        out_shape=[jax.ShapeDtypeStruct((N, 1), jnp.float32),
                   jax.ShapeDtypeStruct((N, 64), jnp.float32),
                   jax.ShapeDtypeStruct((N, 64), jnp.float32)],
    )(h0, dpT)


def _layer_body(alo_ref, ahi_ref, qlo_ref, qhi_ref, s_ref, w_ref, b_ref,
                olo_ref, ohi_ref):
    s = s_ref[...]
    z = jnp.concatenate([alo_ref[...] + qlo_ref[...],
                         ahi_ref[...] + qhi_ref[...]], axis=1) * s
    h = jnp.maximum(jnp.dot(z, w_ref[...]) + b_ref[...], 0.0)
    qn = h * s
    olo_ref[...] = qn[:, :128]
    ohi_ref[...] = qn[:, 128:]


def _tc_layer(alo, ahi, qlo, qhi, s, wm, bias):
    W = alo.shape[1]
    C = 2 * W
    return pl.pallas_call(
        _layer_body,
        grid=(NB,),
        in_specs=[pl.BlockSpec((BLK, W), lambda i: (i, 0)),
                  pl.BlockSpec((BLK, W), lambda i: (i, 0)),
                  pl.BlockSpec((BLK, W), lambda i: (i, 0)),
                  pl.BlockSpec((BLK, W), lambda i: (i, 0)),
                  pl.BlockSpec((BLK, 1), lambda i: (i, 0)),
                  pl.BlockSpec((C, HID), lambda i: (0, 0)),
                  pl.BlockSpec((1, HID), lambda i: (0, 0))],
        out_specs=[pl.BlockSpec((BLK, 128), lambda i: (i, 0)),
                   pl.BlockSpec((BLK, 128), lambda i: (i, 0))],
        out_shape=[jax.ShapeDtypeStruct((N, 128), jnp.float32),
                   jax.ShapeDtypeStruct((N, 128), jnp.float32)],
    )(alo, ahi, qlo, qhi, s, wm, bias.reshape(1, HID))


def _readout_body(alo_ref, ahi_ref, qlo_ref, qhi_ref, s_ref, w_ref, b_ref,
                  bidx_ref, linw_ref, linb_ref, out_ref, sums_scr, cnts_scr):
    i = pl.program_id(0)
    s = s_ref[...]
    z = jnp.concatenate([alo_ref[...] + qlo_ref[...],
                         ahi_ref[...] + qhi_ref[...]], axis=1) * s
    h3 = jnp.maximum(jnp.dot(z, w_ref[...]) + b_ref[...], 0.0)
    gids = lax.broadcasted_iota(jnp.int32, (NG, BLK), 0)
    ohT = (bidx_ref[...] == gids).astype(jnp.float32)
    bs = jnp.dot(ohT, h3)
    bc = jnp.sum(ohT, axis=1, keepdims=True)

    @pl.when(i == 0)
    def _():
        sums_scr[...] = bs
        cnts_scr[...] = bc

    @pl.when(i > 0)
    def _():
        sums_scr[...] += bs
        cnts_scr[...] += bc

    @pl.when(i == NB - 1)
    def _():
        mean = sums_scr[...] / jnp.maximum(cnts_scr[...], 1.0)
        out_ref[...] = jax.nn.sigmoid(jnp.dot(mean, linw_ref[...])
                                      + linb_ref[...])


def _tc_readout(alo, ahi, qlo, qhi, s, wm, bias, bidx_row, lin_W, lin_b):
    return pl.pallas_call(
        _readout_body,
        grid=(NB,),
        in_specs=[pl.BlockSpec((BLK, 128), lambda i: (i, 0)),
                  pl.BlockSpec((BLK, 128), lambda i: (i, 0)),
                  pl.BlockSpec((BLK, 128), lambda i: (i, 0)),
                  pl.BlockSpec((BLK, 128), lambda i: (i, 0)),
                  pl.BlockSpec((BLK, 1), lambda i: (i, 0)),
                  pl.BlockSpec((HID, HID), lambda i: (0, 0)),
                  pl.BlockSpec((1, HID), lambda i: (0, 0)),
                  pl.BlockSpec((1, BLK), lambda i: (0, i)),
                  pl.BlockSpec((HID, 1), lambda i: (0, 0)),
                  pl.BlockSpec((1, 1), lambda i: (0, 0))],
        out_specs=pl.BlockSpec((NG, 1), lambda i: (0, 0)),
        out_shape=jax.ShapeDtypeStruct((NG, 1), jnp.float32),
        scratch_shapes=[pltpu.VMEM((NG, HID), jnp.float32),
                        pltpu.VMEM((NG, 1), jnp.float32)],
    )(alo, ahi, qlo, qhi, s, wm, bias.reshape(1, HID), bidx_row,
      lin_W, lin_b.reshape(1, 1))


def kernel(x, edge_index, batch_idx, atom_emb, W0, b0, W1, b1, W2, b2,
           lin_W, lin_b):
    x = x.astype(jnp.int32)
    src = edge_index[0].astype(jnp.int32)
    dst = edge_index[1].astype(jnp.int32)
    xT = jnp.transpose(x + (jnp.arange(NF, dtype=jnp.int32) * VOCAB)[None, :])
    flat_emb = atom_emb.reshape(NF * VOCAB, EMB)

    h0, degp = _sc_embed_deg(xT, flat_emb, dst)
    s, q0_lo, q0_hi = _tc_prescale(h0, jnp.transpose(degp))

    a_lo, a_hi = _edge_agg_64(src, dst, q0_lo, q0_hi)
    q1_lo, q1_hi = _tc_layer(a_lo, a_hi, q0_lo, q0_hi, s, W0, b0)

    a_lo, a_hi = _edge_agg_128(src, dst, q1_lo, q1_hi)
    q2_lo, q2_hi = _tc_layer(a_lo, a_hi, q1_lo, q1_hi, s, W1, b1)

    a_lo, a_hi = _edge_agg_128(src, dst, q2_lo, q2_hi)
    out = _tc_readout(a_lo, a_hi, q2_lo, q2_hi, s, W2, b2,
                      batch_idx.astype(jnp.int32)[None, :], lin_W, lin_b)
    return out


# speedup vs baseline: 5.1827x; 5.1827x over previous
import jax
import jax.numpy as jnp
from jax import lax
from jax.experimental import pallas as pl
from jax.experimental.pallas import tpu as pltpu
from jax.experimental.pallas import tpu_sc as plsc

N = 10000
E = 320000
NF = 9
VOCAB = 119
EMB = 128
HID = 256
NG = 64

NC = 2
NS = 16
NW = NC * NS

_mesh = plsc.VectorSubcoreMesh(core_axis_name="c", subcore_axis_name="s")


KN = 80
NCHUNK = N // KN
KD = 80
EPT_DEG = E // NW


def _sc_embed_deg_body(xf_hbm, emb_hbm, dst_hbm, h0_hbm, degp_hbm,
                       idxb, gbuf, abuf, oneb, dstb, zb, deg_sh):
    c = lax.axis_index("c")
    s = lax.axis_index("s")
    w = c * NS + s
    for r in range(40):
        zb[pl.ds(r * 16, 16)] = jnp.zeros((16,), jnp.float32)
    r0 = s * 640

    @pl.when(s < 15)
    def _():
        pltpu.sync_copy(zb, deg_sh.at[pl.ds(r0, 640)])

    @pl.when(s == 15)
    def _():
        pltpu.sync_copy(zb.at[pl.ds(0, 400)], deg_sh.at[pl.ds(9600, 400)])

    plsc.subcore_barrier()

    for r in range(5):
        oneb[pl.ds(r * 16, 16)] = jnp.ones((16,), jnp.float32)

    ebase = w * EPT_DEG

    def deg_chunk(k, carry):
        off = ebase + k * KD
        pltpu.sync_copy(dst_hbm.at[pl.ds(off, KD)], dstb)
        pltpu.sync_copy(oneb, deg_sh.at[dstb], add=True)
        return carry

    lax.fori_loop(0, EPT_DEG // KD, deg_chunk, 0)

    for i in range((NCHUNK + NW - 1) // NW):
        cid = w + i * NW

        @pl.when(cid < NCHUNK)
        def _():
            nb = cid * KN
            for f in range(NF):
                pltpu.sync_copy(xf_hbm.at[pl.ds(f * N + nb, KN)], idxb)
                if f == 0:
                    pltpu.sync_copy(emb_hbm.at[idxb], abuf)
                else:
                    pltpu.sync_copy(emb_hbm.at[idxb], gbuf)

                    def addrow(r, carry):
                        for cc in range(EMB // 16):
                            plsc.addupdate(abuf.at[r, pl.ds(cc * 16, 16)],
                                           gbuf[r, pl.ds(cc * 16, 16)])
                        return carry

                    lax.fori_loop(0, KN, addrow, 0)
            pltpu.sync_copy(abuf, h0_hbm.at[pl.ds(nb, KN)])

    plsc.subcore_barrier()

    @pl.when(s < 15)
    def _():
        pltpu.sync_copy(deg_sh.at[pl.ds(r0, 640)], zb)
        pltpu.sync_copy(zb, degp_hbm.at[pl.ds(c * N + r0, 640)])

    @pl.when(s == 15)
    def _():
        pltpu.sync_copy(deg_sh.at[pl.ds(9600, 400)], zb.at[pl.ds(0, 400)])
        pltpu.sync_copy(zb.at[pl.ds(0, 400)],
                        degp_hbm.at[pl.ds(c * N + 9600, 400)])


_sc_embed_deg = pl.kernel(
    _sc_embed_deg_body,
    out_type=[jax.ShapeDtypeStruct((N, EMB), jnp.float32),
              jax.ShapeDtypeStruct((2 * N,), jnp.float32)],
    mesh=_mesh,
    scratch_types=[pltpu.VMEM((KN,), jnp.int32),
                   pltpu.VMEM((KN, EMB), jnp.float32),
                   pltpu.VMEM((KN, EMB), jnp.float32),
                   pltpu.VMEM((KD,), jnp.float32),
                   pltpu.VMEM((KD,), jnp.int32),
                   pltpu.VMEM((640,), jnp.float32),
                   pltpu.VMEM_SHARED((N,), jnp.float32)],
)


QW = 64
KE = 128
EPT = E // NS
NFULL = EPT // KE
KR = EPT - NFULL * KE


def _zero_zb(zb2):
    def zrow(r, carry):
        for cc in range(QW // 16):
            zb2[r, pl.ds(cc * 16, 16)] = jnp.zeros((16,), jnp.float32)
        return carry

    lax.fori_loop(0, 640, zrow, 0)


def _agg_pass(src_hbm, dst_hbm, q_hbm, out_hbm, sb, db, sbr, dbr, gbuf, zb2,
              acc_sh, s):
    _zero_zb(zb2)
    r0 = s * 640

    @pl.when(s < 15)
    def _():
        pltpu.sync_copy(zb2, acc_sh.at[pl.ds(r0, 640)])

    @pl.when(s == 15)
    def _():
        pltpu.sync_copy(zb2.at[pl.ds(0, 400)], acc_sh.at[pl.ds(9600, 400)])

    plsc.subcore_barrier()
    ebase = s * EPT

    def chunk(k, carry):
        off = ebase + k * KE
        pltpu.sync_copy(src_hbm.at[pl.ds(off, KE)], sb)
        pltpu.sync_copy(dst_hbm.at[pl.ds(off, KE)], db)
        pltpu.sync_copy(q_hbm.at[sb], gbuf)
        pltpu.sync_copy(gbuf, acc_sh.at[db], add=True)
        return carry

    lax.fori_loop(0, NFULL, chunk, 0)

    offr = ebase + NFULL * KE
    pltpu.sync_copy(src_hbm.at[pl.ds(offr, KR)], sbr)
    pltpu.sync_copy(dst_hbm.at[pl.ds(offr, KR)], dbr)
    pltpu.sync_copy(q_hbm.at[sbr], gbuf.at[pl.ds(0, KR)])
    pltpu.sync_copy(gbuf.at[pl.ds(0, KR)], acc_sh.at[dbr], add=True)
    plsc.subcore_barrier()

    @pl.when(s < 15)
    def _():
        pltpu.sync_copy(acc_sh.at[pl.ds(r0, 640)], zb2)
        pltpu.sync_copy(zb2, out_hbm.at[pl.ds(r0, 640)])

    @pl.when(s == 15)
    def _():
        pltpu.sync_copy(acc_sh.at[pl.ds(9600, 400)], zb2.at[pl.ds(0, 400)])
        pltpu.sync_copy(zb2.at[pl.ds(0, 400)], out_hbm.at[pl.ds(9600, 400)])


def _edge_body(src_hbm, dst_hbm, q0_hbm, q1_hbm, q2_hbm, q3_hbm,
               a0_hbm, a1_hbm, a2_hbm, a3_hbm,
               sb, db, sbr, dbr, gbuf, zb2, acc_sh):
    c = lax.axis_index("c")
    s = lax.axis_index("s")

    @pl.when(c == 0)
    def _():
        _agg_pass(src_hbm, dst_hbm, q0_hbm, a0_hbm, sb, db, sbr, dbr, gbuf,
                  zb2, acc_sh, s)
        plsc.subcore_barrier()
        _agg_pass(src_hbm, dst_hbm, q1_hbm, a1_hbm, sb, db, sbr, dbr, gbuf,
                  zb2, acc_sh, s)

    @pl.when(c == 1)
    def _():
        _agg_pass(src_hbm, dst_hbm, q2_hbm, a2_hbm, sb, db, sbr, dbr, gbuf,
                  zb2, acc_sh, s)
        plsc.subcore_barrier()
        _agg_pass(src_hbm, dst_hbm, q3_hbm, a3_hbm, sb, db, sbr, dbr, gbuf,
                  zb2, acc_sh, s)


_edge_agg = pl.kernel(
    _edge_body,
    out_type=[jax.ShapeDtypeStruct((N, QW), jnp.float32)] * 4,
    mesh=_mesh,
    scratch_types=[pltpu.VMEM((KE,), jnp.int32),
                   pltpu.VMEM((KE,), jnp.int32),
                   pltpu.VMEM((KR,), jnp.int32),
                   pltpu.VMEM((KR,), jnp.int32),
                   pltpu.VMEM((KE, QW), jnp.float32),
                   pltpu.VMEM((640, QW), jnp.float32),
                   pltpu.VMEM_SHARED((N, QW), jnp.float32)],
    compiler_params=pltpu.CompilerParams(use_tc_tiling_on_sc=False),
)


BLK = 1000
NB = N // BLK


def _prescale_body(h0_ref, dpT_ref, s_ref, qa_ref, qb_ref):
    d = dpT_ref[:, 0:1] + dpT_ref[:, 1:2]
    s = lax.rsqrt(1.0 + d)
    s_ref[...] = s
    q = h0_ref[...] * s
    qa_ref[...] = q[:, :QW]
    qb_ref[...] = q[:, QW:]


def _tc_prescale(h0, dpT):
    return pl.pallas_call(
        _prescale_body,
        grid=(NB,),
        in_specs=[pl.BlockSpec((BLK, EMB), lambda i: (i, 0)),
                  pl.BlockSpec((BLK, 2), lambda i: (i, 0))],
        out_specs=[pl.BlockSpec((BLK, 1), lambda i: (i, 0)),
                   pl.BlockSpec((BLK, QW), lambda i: (i, 0)),
                   pl.BlockSpec((BLK, QW), lambda i: (i, 0))],
        out_shape=[jax.ShapeDtypeStruct((N, 1), jnp.float32),
                   jax.ShapeDtypeStruct((N, QW), jnp.float32),
                   jax.ShapeDtypeStruct((N, QW), jnp.float32)],
    )(h0, dpT)


def _layer_body(a0, a1, a2, a3, q0, q1, q2, q3, s_ref, w_ref, b_ref,
                o0, o1, o2, o3):
    s = s_ref[...]
    z = jnp.concatenate([a0[...] + q0[...], a1[...] + q1[...],
                         a2[...] + q2[...], a3[...] + q3[...]], axis=1) * s
    h = jnp.maximum(jnp.dot(z, w_ref[...]) + b_ref[...], 0.0)
    qn = h * s
    o0[...] = qn[:, 0 * QW:1 * QW]
    o1[...] = qn[:, 1 * QW:2 * QW]
    o2[...] = qn[:, 2 * QW:3 * QW]
    o3[...] = qn[:, 3 * QW:4 * QW]


_qspec = pl.BlockSpec((BLK, QW), lambda i: (i, 0))


def _tc_layer(aq, qq, s, wm, bias):
    return pl.pallas_call(
        _layer_body,
        grid=(NB,),
        in_specs=[_qspec] * 8 + [
            pl.BlockSpec((BLK, 1), lambda i: (i, 0)),
            pl.BlockSpec((HID, HID), lambda i: (0, 0)),
            pl.BlockSpec((1, HID), lambda i: (0, 0))],
        out_specs=[_qspec] * 4,
        out_shape=[jax.ShapeDtypeStruct((N, QW), jnp.float32)] * 4,
    )(*aq, *qq, s, wm, bias.reshape(1, HID))


def _readout_body(q0, q1, q2, q3, s_ref, bidx_ref, linw_ref, linb_ref,
                  out_ref, sums_scr, cnts_scr):
    i = pl.program_id(0)
    s = s_ref[...]
    h3 = jnp.concatenate([q0[...], q1[...], q2[...], q3[...]], axis=1) / s
    gids = lax.broadcasted_iota(jnp.int32, (NG, BLK), 0)
    ohT = (bidx_ref[0] == gids).astype(jnp.float32)
    bs = jnp.dot(ohT, h3)
    bc = jnp.sum(ohT, axis=1, keepdims=True)

    @pl.when(i == 0)
    def _():
        sums_scr[...] = bs
        cnts_scr[...] = bc

    @pl.when(i > 0)
    def _():
        sums_scr[...] += bs
        cnts_scr[...] += bc

    @pl.when(i == NB - 1)
    def _():
        mean = sums_scr[...] / jnp.maximum(cnts_scr[...], 1.0)
        out_ref[...] = jax.nn.sigmoid(jnp.dot(mean, linw_ref[...])
                                      + linb_ref[...])


def _tc_readout(qq, s, bidx_3d, lin_W, lin_b):
    return pl.pallas_call(
        _readout_body,
        grid=(NB,),
        in_specs=[_qspec] * 4 + [
            pl.BlockSpec((BLK, 1), lambda i: (i, 0)),
            pl.BlockSpec((1, 1, BLK), lambda i: (i, 0, 0)),
            pl.BlockSpec((HID, 1), lambda i: (0, 0)),
            pl.BlockSpec((1, 1), lambda i: (0, 0))],
        out_specs=pl.BlockSpec((NG, 1), lambda i: (0, 0)),
        out_shape=jax.ShapeDtypeStruct((NG, 1), jnp.float32),
        scratch_shapes=[pltpu.VMEM((NG, HID), jnp.float32),
                        pltpu.VMEM((NG, 1), jnp.float32)],
    )(*qq, s, bidx_3d, lin_W, lin_b.reshape(1, 1))


def kernel(x, edge_index, batch_idx, atom_emb, W0, b0, W1, b1, W2, b2,
           lin_W, lin_b):
    x = x.astype(jnp.int32)
    src = edge_index[0].astype(jnp.int32)
    dst = edge_index[1].astype(jnp.int32)
    xf = jnp.transpose(
        x + (jnp.arange(NF, dtype=jnp.int32) * VOCAB)[None, :]).reshape(-1)
    flat_emb = atom_emb.reshape(NF * VOCAB, EMB)

    h0, degp = _sc_embed_deg(xf, flat_emb, dst)
    s, q0a, q0b = _tc_prescale(h0, jnp.transpose(degp.reshape(2, N)))

    w0p = jnp.concatenate([W0, jnp.zeros((HID - EMB, HID), jnp.float32)], 0)
    wstack = jnp.stack([w0p, W1, W2])
    bstack = jnp.stack([b0, b1, b2])
    zq = jnp.zeros((N, QW), jnp.float32)

    def _layer_step(carry, wb):
        wm, bias = wb
        aq = _edge_agg(src, dst, *carry)
        nq = _tc_layer(aq, carry, s, wm, bias)
        return tuple(nq), None

    q3, _ = lax.scan(_layer_step, (q0a, q0b, zq, zq), (wstack, bstack))

    out = _tc_readout(q3, s,
                      batch_idx.astype(jnp.int32).reshape(NB, 1, BLK),
                      lin_W, lin_b)
    return out


# speedup vs baseline: 10.1666x; 1.9616x over previous
import jax
import jax.numpy as jnp
from jax import lax
from jax.experimental import pallas as pl
from jax.experimental.pallas import tpu as pltpu
from jax.experimental.pallas import tpu_sc as plsc

N = 10000
E = 320000
NF = 9
VOCAB = 119
EMB = 128
HID = 256
NG = 64

NC = 2
NS = 16
NW = NC * NS

_mesh = plsc.VectorSubcoreMesh(core_axis_name="c", subcore_axis_name="s")


KN = 80
NCHUNK = N // KN
KD = 80
EPT_DEG = E // NW


def _sc_embed_deg_body(xf_hbm, emb_hbm, dst_hbm, h0_hbm, degp_hbm,
                       idxb, gbuf, abuf, oneb, dstb, zb, deg_sh):
    c = lax.axis_index("c")
    s = lax.axis_index("s")
    w = c * NS + s
    for r in range(40):
        zb[pl.ds(r * 16, 16)] = jnp.zeros((16,), jnp.float32)
    r0 = s * 640

    @pl.when(s < 15)
    def _():
        pltpu.sync_copy(zb, deg_sh.at[pl.ds(r0, 640)])

    @pl.when(s == 15)
    def _():
        pltpu.sync_copy(zb.at[pl.ds(0, 400)], deg_sh.at[pl.ds(9600, 400)])

    plsc.subcore_barrier()

    for r in range(5):
        oneb[pl.ds(r * 16, 16)] = jnp.ones((16,), jnp.float32)

    ebase = w * EPT_DEG

    def deg_chunk(k, carry):
        off = ebase + k * KD
        pltpu.sync_copy(dst_hbm.at[pl.ds(off, KD)], dstb)
        pltpu.sync_copy(oneb, deg_sh.at[dstb], add=True)
        return carry

    lax.fori_loop(0, EPT_DEG // KD, deg_chunk, 0)

    for i in range((NCHUNK + NW - 1) // NW):
        cid = w + i * NW

        @pl.when(cid < NCHUNK)
        def _():
            nb = cid * KN
            for f in range(NF):
                pltpu.sync_copy(xf_hbm.at[pl.ds(f * N + nb, KN)], idxb)
                if f == 0:
                    pltpu.sync_copy(emb_hbm.at[idxb], abuf)
                else:
                    pltpu.sync_copy(emb_hbm.at[idxb], gbuf)

                    def addrow(r, carry):
                        for cc in range(EMB // 16):
                            plsc.addupdate(abuf.at[r, pl.ds(cc * 16, 16)],
                                           gbuf[r, pl.ds(cc * 16, 16)])
                        return carry

                    lax.fori_loop(0, KN, addrow, 0)
            pltpu.sync_copy(abuf, h0_hbm.at[pl.ds(nb, KN)])

    plsc.subcore_barrier()

    @pl.when(s < 15)
    def _():
        pltpu.sync_copy(deg_sh.at[pl.ds(r0, 640)], zb)
        pltpu.sync_copy(zb, degp_hbm.at[pl.ds(c * N + r0, 640)])

    @pl.when(s == 15)
    def _():
        pltpu.sync_copy(deg_sh.at[pl.ds(9600, 400)], zb.at[pl.ds(0, 400)])
        pltpu.sync_copy(zb.at[pl.ds(0, 400)],
                        degp_hbm.at[pl.ds(c * N + 9600, 400)])


_sc_embed_deg = pl.kernel(
    _sc_embed_deg_body,
    out_type=[jax.ShapeDtypeStruct((N, EMB), jnp.float32),
              jax.ShapeDtypeStruct((2 * N,), jnp.float32)],
    mesh=_mesh,
    scratch_types=[pltpu.VMEM((KN,), jnp.int32),
                   pltpu.VMEM((KN, EMB), jnp.float32),
                   pltpu.VMEM((KN, EMB), jnp.float32),
                   pltpu.VMEM((KD,), jnp.float32),
                   pltpu.VMEM((KD,), jnp.int32),
                   pltpu.VMEM((640,), jnp.float32),
                   pltpu.VMEM_SHARED((N,), jnp.float32)],
)


QW = 64
KE = 128
EPT = E // NS
NCH = (EPT + KE - 1) // KE
EPAD = NCH * KE - EPT
NBUF = 4
NGRP = NCH // NBUF
DUMP = N


def _zero_g0(g0):
    def zrow(r, carry):
        for cc in range(QW // 16):
            g0[r, pl.ds(cc * 16, 16)] = jnp.zeros((16,), jnp.float32)
        return carry

    lax.fori_loop(0, KE, zrow, 0)


def _acc_rows(s, fn):
    r0 = s * 640

    @pl.when(s < 15)
    def _():
        for i in range(5):
            fn(r0 + i * KE, KE)

    @pl.when(s == 15)
    def _():
        for i in range(3):
            fn(9600 + i * KE, KE)
        fn(9984, 16)


def _agg_pass(q_hbm, out_hbm, sb2, db2, gbufs, gsems, ssems, acc_sh, s):
    g0 = gbufs[0]
    _zero_g0(g0)
    _acc_rows(s, lambda r, n: pltpu.sync_copy(
        g0.at[pl.ds(0, n)], acc_sh.at[pl.ds(r, n)]))
    plsc.subcore_barrier()

    for b in range(NBUF):
        pltpu.async_copy(q_hbm.at[sb2.at[b]], gbufs[b], gsems[b])

    def group(j, carry):
        descs = []
        for b in range(NBUF):
            k = NBUF * j + b
            pltpu.make_async_copy(q_hbm.at[sb2.at[0]], gbufs[b],
                                  gsems[b]).wait()
            descs.append(pltpu.async_copy(
                gbufs[b], acc_sh.at[db2.at[k]], ssems[b], add=True))
        for b in range(NBUF):
            k = NBUF * j + b
            descs[b].wait()

            @pl.when(k + NBUF < NCH)
            def _(b=b, k=k):
                pltpu.async_copy(q_hbm.at[sb2.at[k + NBUF]], gbufs[b],
                                 gsems[b])
        return carry

    lax.fori_loop(0, NGRP, group, 0)
    pltpu.make_async_copy(q_hbm.at[sb2.at[0]], gbufs[0], gsems[0]).wait()
    pltpu.async_copy(gbufs[0], acc_sh.at[db2.at[NCH - 1]], ssems[0],
                     add=True).wait()
    plsc.subcore_barrier()

    def wb(r, n):
        pltpu.sync_copy(acc_sh.at[pl.ds(r, n)], g0.at[pl.ds(0, n)])
        pltpu.sync_copy(g0.at[pl.ds(0, n)], out_hbm.at[pl.ds(r, n)])

    _acc_rows(s, wb)


def _edge_body(src3d_hbm, dst3d_hbm, q0_hbm, q1_hbm, q2_hbm, q3_hbm,
               a0_hbm, a1_hbm, a2_hbm, a3_hbm,
               sb2, db2, g0, g1, g2, g3, acc_sh,
               gs0, gs1, gs2, gs3, ss0, ss1, ss2, ss3):
    c = lax.axis_index("c")
    s = lax.axis_index("s")
    gbufs = (g0, g1, g2, g3)
    gsems = (gs0, gs1, gs2, gs3)
    ssems = (ss0, ss1, ss2, ss3)
    pltpu.sync_copy(src3d_hbm.at[s], sb2)
    pltpu.sync_copy(dst3d_hbm.at[s], db2)

    @pl.when(c == 0)
    def _():
        _agg_pass(q0_hbm, a0_hbm, sb2, db2, gbufs, gsems, ssems, acc_sh, s)
        plsc.subcore_barrier()
        _agg_pass(q1_hbm, a1_hbm, sb2, db2, gbufs, gsems, ssems, acc_sh, s)

    @pl.when(c == 1)
    def _():
        _agg_pass(q2_hbm, a2_hbm, sb2, db2, gbufs, gsems, ssems, acc_sh, s)
        plsc.subcore_barrier()
        _agg_pass(q3_hbm, a3_hbm, sb2, db2, gbufs, gsems, ssems, acc_sh, s)


_edge_agg = pl.kernel(
    _edge_body,
    out_type=[jax.ShapeDtypeStruct((N, QW), jnp.float32)] * 4,
    mesh=_mesh,
    scratch_types=[pltpu.VMEM((NCH, KE), jnp.int32),
                   pltpu.VMEM((NCH, KE), jnp.int32),
                   pltpu.VMEM((KE, QW), jnp.float32),
                   pltpu.VMEM((KE, QW), jnp.float32),
                   pltpu.VMEM((KE, QW), jnp.float32),
                   pltpu.VMEM((KE, QW), jnp.float32),
                   pltpu.VMEM_SHARED((N + 16, QW), jnp.float32),
                   pltpu.SemaphoreType.DMA,
                   pltpu.SemaphoreType.DMA,
                   pltpu.SemaphoreType.DMA,
                   pltpu.SemaphoreType.DMA,
                   pltpu.SemaphoreType.DMA,
                   pltpu.SemaphoreType.DMA,
                   pltpu.SemaphoreType.DMA,
                   pltpu.SemaphoreType.DMA],
    compiler_params=pltpu.CompilerParams(use_tc_tiling_on_sc=False),
)


BLK = 1000
NB = N // BLK


def _prescale_body(h0_ref, dpT_ref, s_ref, qa_ref, qb_ref):
    d = dpT_ref[:, 0:1] + dpT_ref[:, 1:2]
    s = lax.rsqrt(1.0 + d)
    s_ref[...] = s
    q = h0_ref[...] * s
    qa_ref[...] = q[:, :QW]
    qb_ref[...] = q[:, QW:]


def _tc_prescale(h0, dpT):
    return pl.pallas_call(
        _prescale_body,
        grid=(NB,),
        in_specs=[pl.BlockSpec((BLK, EMB), lambda i: (i, 0)),
                  pl.BlockSpec((BLK, 2), lambda i: (i, 0))],
        out_specs=[pl.BlockSpec((BLK, 1), lambda i: (i, 0)),
                   pl.BlockSpec((BLK, QW), lambda i: (i, 0)),
                   pl.BlockSpec((BLK, QW), lambda i: (i, 0))],
        out_shape=[jax.ShapeDtypeStruct((N, 1), jnp.float32),
                   jax.ShapeDtypeStruct((N, QW), jnp.float32),
                   jax.ShapeDtypeStruct((N, QW), jnp.float32)],
    )(h0, dpT)


def _layer_body(a0, a1, a2, a3, q0, q1, q2, q3, s_ref, w_ref, b_ref,
                o0, o1, o2, o3):
    s = s_ref[...]
    z = jnp.concatenate([a0[...] + q0[...], a1[...] + q1[...],
                         a2[...] + q2[...], a3[...] + q3[...]], axis=1) * s
    h = jnp.maximum(jnp.dot(z, w_ref[...]) + b_ref[...], 0.0)
    qn = h * s
    o0[...] = qn[:, 0 * QW:1 * QW]
    o1[...] = qn[:, 1 * QW:2 * QW]
    o2[...] = qn[:, 2 * QW:3 * QW]
    o3[...] = qn[:, 3 * QW:4 * QW]


_qspec = pl.BlockSpec((BLK, QW), lambda i: (i, 0))


def _tc_layer(aq, qq, s, wm, bias):
    return pl.pallas_call(
        _layer_body,
        grid=(NB,),
        in_specs=[_qspec] * 8 + [
            pl.BlockSpec((BLK, 1), lambda i: (i, 0)),
            pl.BlockSpec((HID, HID), lambda i: (0, 0)),
            pl.BlockSpec((1, HID), lambda i: (0, 0))],
        out_specs=[_qspec] * 4,
        out_shape=[jax.ShapeDtypeStruct((N, QW), jnp.float32)] * 4,
    )(*aq, *qq, s, wm, bias.reshape(1, HID))


def _readout_body(q0, q1, q2, q3, s_ref, bidx_ref, linw_ref, linb_ref,
                  out_ref, sums_scr, cnts_scr):
    i = pl.program_id(0)
    s = s_ref[...]
    h3 = jnp.concatenate([q0[...], q1[...], q2[...], q3[...]], axis=1) / s
    gids = lax.broadcasted_iota(jnp.int32, (NG, BLK), 0)
    ohT = (bidx_ref[0] == gids).astype(jnp.float32)
    bs = jnp.dot(ohT, h3)
    bc = jnp.sum(ohT, axis=1, keepdims=True)

    @pl.when(i == 0)
    def _():
        sums_scr[...] = bs
        cnts_scr[...] = bc

    @pl.when(i > 0)
    def _():
        sums_scr[...] += bs
        cnts_scr[...] += bc

    @pl.when(i == NB - 1)
    def _():
        mean = sums_scr[...] / jnp.maximum(cnts_scr[...], 1.0)
        out_ref[...] = jax.nn.sigmoid(jnp.dot(mean, linw_ref[...])
                                      + linb_ref[...])


def _tc_readout(qq, s, bidx_3d, lin_W, lin_b):
    return pl.pallas_call(
        _readout_body,
        grid=(NB,),
        in_specs=[_qspec] * 4 + [
            pl.BlockSpec((BLK, 1), lambda i: (i, 0)),
            pl.BlockSpec((1, 1, BLK), lambda i: (i, 0, 0)),
            pl.BlockSpec((HID, 1), lambda i: (0, 0)),
            pl.BlockSpec((1, 1), lambda i: (0, 0))],
        out_specs=pl.BlockSpec((NG, 1), lambda i: (0, 0)),
        out_shape=jax.ShapeDtypeStruct((NG, 1), jnp.float32),
        scratch_shapes=[pltpu.VMEM((NG, HID), jnp.float32),
                        pltpu.VMEM((NG, 1), jnp.float32)],
    )(*qq, s, bidx_3d, lin_W, lin_b.reshape(1, 1))


def kernel(x, edge_index, batch_idx, atom_emb, W0, b0, W1, b1, W2, b2,
           lin_W, lin_b):
    x = x.astype(jnp.int32)
    src = edge_index[0].astype(jnp.int32)
    dst = edge_index[1].astype(jnp.int32)
    xf = jnp.transpose(
        x + (jnp.arange(NF, dtype=jnp.int32) * VOCAB)[None, :]).reshape(-1)
    flat_emb = atom_emb.reshape(NF * VOCAB, EMB)

    h0, degp = _sc_embed_deg(xf, flat_emb, dst)
    s, q0a, q0b = _tc_prescale(h0, jnp.transpose(degp.reshape(2, N)))

    w0p = jnp.concatenate([W0, jnp.zeros((HID - EMB, HID), jnp.float32)], 0)
    wstack = jnp.stack([w0p, W1, W2])
    bstack = jnp.stack([b0, b1, b2])
    zq = jnp.zeros((N, QW), jnp.float32)

    srcr = src.reshape(NS, EPT)
    dstr = dst.reshape(NS, EPT)
    src3d = jnp.concatenate(
        [srcr, jnp.zeros((NS, EPAD), jnp.int32)], axis=1).reshape(NS, NCH, KE)
    dst3d = jnp.concatenate(
        [dstr, jnp.full((NS, EPAD), DUMP, jnp.int32)],
        axis=1).reshape(NS, NCH, KE)

    def _layer_step(carry, wb):
        wm, bias = wb
        aq = _edge_agg(src3d, dst3d, *carry)
        nq = _tc_layer(aq, carry, s, wm, bias)
        return tuple(nq), None

    q3, _ = lax.scan(_layer_step, (q0a, q0b, zq, zq), (wstack, bstack))

    out = _tc_readout(q3, s,
                      batch_idx.astype(jnp.int32).reshape(NB, 1, BLK),
                      lin_W, lin_b)
    return out


# speedup vs baseline: 10.4968x; 1.0325x over previous
import jax
import jax.numpy as jnp
from jax import lax
from jax.experimental import pallas as pl
from jax.experimental.pallas import tpu as pltpu
from jax.experimental.pallas import tpu_sc as plsc

N = 10000
E = 320000
NF = 9
VOCAB = 119
EMB = 128
HID = 256
NG = 64

NC = 2
NS = 16
NW = NC * NS

_mesh = plsc.VectorSubcoreMesh(core_axis_name="c", subcore_axis_name="s")


KN = 80
NCHUNK = N // KN


def _sc_embed_deg_body(xg_hbm, emb_hbm, dst3d_hbm, h0_hbm, degp_hbm,
                       idxb, gA, gB, abuf, oneb, db2, zb, deg_sh,
                       sem_a, sg0, sg1, sd0, sd1):
    c = lax.axis_index("c")
    s = lax.axis_index("s")
    w = c * NS + s
    for r in range(40):
        zb[pl.ds(r * 16, 16)] = jnp.zeros((16,), jnp.float32)
    r0 = s * 640

    @pl.when(s < 15)
    def _():
        pltpu.sync_copy(zb, deg_sh.at[pl.ds(r0, 640)])

    @pl.when(s == 15)
    def _():
        pltpu.sync_copy(zb.at[pl.ds(0, 400)], deg_sh.at[pl.ds(9600, 400)])

    for r in range(8):
        oneb[pl.ds(r * 16, 16)] = jnp.ones((16,), jnp.float32)

    pltpu.sync_copy(dst3d_hbm.at[s], db2)
    plsc.subcore_barrier()

    kbase = c * 79

    def deg_pair(j, carry):
        k0 = kbase + 2 * j
        d0 = pltpu.async_copy(oneb, deg_sh.at[db2.at[k0]], sd0, add=True)
        d1 = pltpu.async_copy(oneb, deg_sh.at[db2.at[k0 + 1]], sd1, add=True)
        d0.wait()
        d1.wait()
        return carry

    lax.fori_loop(0, 39, deg_pair, 0)

    @pl.when(c == 0)
    def _():
        pltpu.sync_copy(oneb, deg_sh.at[db2.at[78]], add=True)

    rings = (gA, gB)
    rsems = (sg0, sg1)
    for i in range((NCHUNK + NW - 1) // NW):
        cid = w + i * NW

        @pl.when(cid < NCHUNK)
        def _(cid=cid):
            pltpu.sync_copy(xg_hbm.at[cid], idxb)
            da = pltpu.async_copy(emb_hbm.at[idxb.at[0]], abuf, sem_a)
            dg = pltpu.async_copy(emb_hbm.at[idxb.at[1]], rings[0], rsems[0])
            da.wait()
            for f in range(1, NF):
                dg.wait()
                rb = rings[(f - 1) % 2]
                if f + 1 < NF:
                    dg = pltpu.async_copy(emb_hbm.at[idxb.at[f + 1]],
                                          rings[f % 2], rsems[f % 2])

                def addrow(r, carry, rb=rb):
                    for cc in range(EMB // 16):
                        plsc.addupdate(abuf.at[r, pl.ds(cc * 16, 16)],
                                       rb[r, pl.ds(cc * 16, 16)])
                    return carry

                lax.fori_loop(0, KN, addrow, 0)
            pltpu.sync_copy(abuf, h0_hbm.at[pl.ds(cid * KN, KN)])

    plsc.subcore_barrier()

    @pl.when(s < 15)
    def _():
        pltpu.sync_copy(deg_sh.at[pl.ds(r0, 640)], zb)
        pltpu.sync_copy(zb, degp_hbm.at[pl.ds(c * N + r0, 640)])

    @pl.when(s == 15)
    def _():
        pltpu.sync_copy(deg_sh.at[pl.ds(9600, 400)], zb.at[pl.ds(0, 400)])
        pltpu.sync_copy(zb.at[pl.ds(0, 400)],
                        degp_hbm.at[pl.ds(c * N + 9600, 400)])


_sc_embed_deg = pl.kernel(
    _sc_embed_deg_body,
    out_type=[jax.ShapeDtypeStruct((N, EMB), jnp.float32),
              jax.ShapeDtypeStruct((2 * N,), jnp.float32)],
    mesh=_mesh,
    scratch_types=[pltpu.VMEM((NF, KN), jnp.int32),
                   pltpu.VMEM((KN, EMB), jnp.float32),
                   pltpu.VMEM((KN, EMB), jnp.float32),
                   pltpu.VMEM((KN, EMB), jnp.float32),
                   pltpu.VMEM((128,), jnp.float32),
                   pltpu.VMEM((157, 128), jnp.int32),
                   pltpu.VMEM((640,), jnp.float32),
                   pltpu.VMEM_SHARED((N + 16,), jnp.float32),
                   pltpu.SemaphoreType.DMA,
                   pltpu.SemaphoreType.DMA,
                   pltpu.SemaphoreType.DMA,
                   pltpu.SemaphoreType.DMA,
                   pltpu.SemaphoreType.DMA],
    compiler_params=pltpu.CompilerParams(use_tc_tiling_on_sc=False),
)


QW = 64
KE = 128
EPT = E // NS
NCH = (EPT + KE - 1) // KE
EPAD = NCH * KE - EPT
NBUF = 4
NGRP = NCH // NBUF
DUMP = N


def _zero_g0(g0):
    def zrow(r, carry):
        for cc in range(QW // 16):
            g0[r, pl.ds(cc * 16, 16)] = jnp.zeros((16,), jnp.float32)
        return carry

    lax.fori_loop(0, KE, zrow, 0)


def _acc_rows(s, fn):
    r0 = s * 640

    @pl.when(s < 15)
    def _():
        for i in range(5):
            fn(r0 + i * KE, KE)

    @pl.when(s == 15)
    def _():
        for i in range(3):
            fn(9600 + i * KE, KE)
        fn(9984, 16)


def _agg_pass(q_hbm, out_hbm, sb2, db2, gbufs, gsems, ssems, acc_sh, s):
    g0 = gbufs[0]
    _zero_g0(g0)
    _acc_rows(s, lambda r, n: pltpu.sync_copy(
        g0.at[pl.ds(0, n)], acc_sh.at[pl.ds(r, n)]))
    plsc.subcore_barrier()

    for b in range(NBUF):
        pltpu.async_copy(q_hbm.at[sb2.at[b]], gbufs[b], gsems[b])

    def group(j, carry):
        descs = []
        for b in range(NBUF):
            k = NBUF * j + b
            pltpu.make_async_copy(q_hbm.at[sb2.at[0]], gbufs[b],
                                  gsems[b]).wait()
            descs.append(pltpu.async_copy(
                gbufs[b], acc_sh.at[db2.at[k]], ssems[b], add=True))
        for b in range(NBUF):
            k = NBUF * j + b
            descs[b].wait()

            @pl.when(k + NBUF < NCH)
            def _(b=b, k=k):
                pltpu.async_copy(q_hbm.at[sb2.at[k + NBUF]], gbufs[b],
                                 gsems[b])
        return carry

    lax.fori_loop(0, NGRP, group, 0)
    pltpu.make_async_copy(q_hbm.at[sb2.at[0]], gbufs[0], gsems[0]).wait()
    pltpu.async_copy(gbufs[0], acc_sh.at[db2.at[NCH - 1]], ssems[0],
                     add=True).wait()
    plsc.subcore_barrier()

    def wb(r, n):
        pltpu.sync_copy(acc_sh.at[pl.ds(r, n)], g0.at[pl.ds(0, n)])
        pltpu.sync_copy(g0.at[pl.ds(0, n)], out_hbm.at[pl.ds(r, n)])

    _acc_rows(s, wb)


def _edge_body(src3d_hbm, dst3d_hbm, q0_hbm, q1_hbm, q2_hbm, q3_hbm,
               a0_hbm, a1_hbm, a2_hbm, a3_hbm,
               sb2, db2, g0, g1, g2, g3, acc_sh,
               gs0, gs1, gs2, gs3, ss0, ss1, ss2, ss3):
    c = lax.axis_index("c")
    s = lax.axis_index("s")
    gbufs = (g0, g1, g2, g3)
    gsems = (gs0, gs1, gs2, gs3)
    ssems = (ss0, ss1, ss2, ss3)
    pltpu.sync_copy(src3d_hbm.at[s], sb2)
    pltpu.sync_copy(dst3d_hbm.at[s], db2)

    @pl.when(c == 0)
    def _():
        _agg_pass(q0_hbm, a0_hbm, sb2, db2, gbufs, gsems, ssems, acc_sh, s)
        plsc.subcore_barrier()
        _agg_pass(q1_hbm, a1_hbm, sb2, db2, gbufs, gsems, ssems, acc_sh, s)

    @pl.when(c == 1)
    def _():
        _agg_pass(q2_hbm, a2_hbm, sb2, db2, gbufs, gsems, ssems, acc_sh, s)
        plsc.subcore_barrier()
        _agg_pass(q3_hbm, a3_hbm, sb2, db2, gbufs, gsems, ssems, acc_sh, s)


_edge_agg = pl.kernel(
    _edge_body,
    out_type=[jax.ShapeDtypeStruct((N, QW), jnp.float32)] * 4,
    mesh=_mesh,
    scratch_types=[pltpu.VMEM((NCH, KE), jnp.int32),
                   pltpu.VMEM((NCH, KE), jnp.int32),
                   pltpu.VMEM((KE, QW), jnp.float32),
                   pltpu.VMEM((KE, QW), jnp.float32),
                   pltpu.VMEM((KE, QW), jnp.float32),
                   pltpu.VMEM((KE, QW), jnp.float32),
                   pltpu.VMEM_SHARED((N + 16, QW), jnp.float32),
                   pltpu.SemaphoreType.DMA,
                   pltpu.SemaphoreType.DMA,
                   pltpu.SemaphoreType.DMA,
                   pltpu.SemaphoreType.DMA,
                   pltpu.SemaphoreType.DMA,
                   pltpu.SemaphoreType.DMA,
                   pltpu.SemaphoreType.DMA,
                   pltpu.SemaphoreType.DMA],
    compiler_params=pltpu.CompilerParams(use_tc_tiling_on_sc=False),
)


BLK = 1000
NB = N // BLK


def _prescale_body(h0_ref, dpT_ref, s_ref, qa_ref, qb_ref):
    d = dpT_ref[:, 0:1] + dpT_ref[:, 1:2]
    s = lax.rsqrt(1.0 + d)
    s_ref[...] = s
    q = h0_ref[...] * s
    qa_ref[...] = q[:, :QW]
    qb_ref[...] = q[:, QW:]


def _tc_prescale(h0, dpT):
    return pl.pallas_call(
        _prescale_body,
        grid=(NB,),
        in_specs=[pl.BlockSpec((BLK, EMB), lambda i: (i, 0)),
                  pl.BlockSpec((BLK, 2), lambda i: (i, 0))],
        out_specs=[pl.BlockSpec((BLK, 1), lambda i: (i, 0)),
                   pl.BlockSpec((BLK, QW), lambda i: (i, 0)),
                   pl.BlockSpec((BLK, QW), lambda i: (i, 0))],
        out_shape=[jax.ShapeDtypeStruct((N, 1), jnp.float32),
                   jax.ShapeDtypeStruct((N, QW), jnp.float32),
                   jax.ShapeDtypeStruct((N, QW), jnp.float32)],
    )(h0, dpT)


def _layer_body(a0, a1, a2, a3, q0, q1, q2, q3, s_ref, w_ref, b_ref,
                o0, o1, o2, o3):
    s = s_ref[...]
    z = jnp.concatenate([a0[...] + q0[...], a1[...] + q1[...],
                         a2[...] + q2[...], a3[...] + q3[...]], axis=1) * s
    h = jnp.maximum(jnp.dot(z, w_ref[...]) + b_ref[...], 0.0)
    qn = h * s
    o0[...] = qn[:, 0 * QW:1 * QW]
    o1[...] = qn[:, 1 * QW:2 * QW]
    o2[...] = qn[:, 2 * QW:3 * QW]
    o3[...] = qn[:, 3 * QW:4 * QW]


_qspec = pl.BlockSpec((BLK, QW), lambda i: (i, 0))


def _tc_layer(aq, qq, s, wm, bias):
    return pl.pallas_call(
        _layer_body,
        grid=(NB,),
        in_specs=[_qspec] * 8 + [
            pl.BlockSpec((BLK, 1), lambda i: (i, 0)),
            pl.BlockSpec((HID, HID), lambda i: (0, 0)),
            pl.BlockSpec((1, HID), lambda i: (0, 0))],
        out_specs=[_qspec] * 4,
        out_shape=[jax.ShapeDtypeStruct((N, QW), jnp.float32)] * 4,
    )(*aq, *qq, s, wm, bias.reshape(1, HID))


def _readout_body(q0, q1, q2, q3, s_ref, bidx_ref, linw_ref, linb_ref,
                  out_ref, sums_scr, cnts_scr):
    i = pl.program_id(0)
    s = s_ref[...]
    h3 = jnp.concatenate([q0[...], q1[...], q2[...], q3[...]], axis=1) / s
    gids = lax.broadcasted_iota(jnp.int32, (NG, BLK), 0)
    ohT = (bidx_ref[0] == gids).astype(jnp.float32)
    bs = jnp.dot(ohT, h3)
    bc = jnp.sum(ohT, axis=1, keepdims=True)

    @pl.when(i == 0)
    def _():
        sums_scr[...] = bs
        cnts_scr[...] = bc

    @pl.when(i > 0)
    def _():
        sums_scr[...] += bs
        cnts_scr[...] += bc

    @pl.when(i == NB - 1)
    def _():
        mean = sums_scr[...] / jnp.maximum(cnts_scr[...], 1.0)
        out_ref[...] = jax.nn.sigmoid(jnp.dot(mean, linw_ref[...])
                                      + linb_ref[...])


def _tc_readout(qq, s, bidx_3d, lin_W, lin_b):
    return pl.pallas_call(
        _readout_body,
        grid=(NB,),
        in_specs=[_qspec] * 4 + [
            pl.BlockSpec((BLK, 1), lambda i: (i, 0)),
            pl.BlockSpec((1, 1, BLK), lambda i: (i, 0, 0)),
            pl.BlockSpec((HID, 1), lambda i: (0, 0)),
            pl.BlockSpec((1, 1), lambda i: (0, 0))],
        out_specs=pl.BlockSpec((NG, 1), lambda i: (0, 0)),
        out_shape=jax.ShapeDtypeStruct((NG, 1), jnp.float32),
        scratch_shapes=[pltpu.VMEM((NG, HID), jnp.float32),
                        pltpu.VMEM((NG, 1), jnp.float32)],
    )(*qq, s, bidx_3d, lin_W, lin_b.reshape(1, 1))


def kernel(x, edge_index, batch_idx, atom_emb, W0, b0, W1, b1, W2, b2,
           lin_W, lin_b):
    x = x.astype(jnp.int32)
    src = edge_index[0].astype(jnp.int32)
    dst = edge_index[1].astype(jnp.int32)
    xg = jnp.transpose(
        x + (jnp.arange(NF, dtype=jnp.int32) * VOCAB)[None, :]
    ).reshape(NF, NCHUNK, KN).transpose(1, 0, 2)
    flat_emb = atom_emb.reshape(NF * VOCAB, EMB)

    srcr = src.reshape(NS, EPT)
    dstr = dst.reshape(NS, EPT)
    src3d = jnp.concatenate(
        [srcr, jnp.zeros((NS, EPAD), jnp.int32)], axis=1).reshape(NS, NCH, KE)
    dst3d = jnp.concatenate(
        [dstr, jnp.full((NS, EPAD), DUMP, jnp.int32)],
        axis=1).reshape(NS, NCH, KE)

    h0, degp = _sc_embed_deg(xg, flat_emb, dst3d)
    s, q0a, q0b = _tc_prescale(h0, jnp.transpose(degp.reshape(2, N)))

    w0p = jnp.concatenate([W0, jnp.zeros((HID - EMB, HID), jnp.float32)], 0)
    wstack = jnp.stack([w0p, W1, W2])
    bstack = jnp.stack([b0, b1, b2])
    zq = jnp.zeros((N, QW), jnp.float32)

    def _layer_step(carry, wb):
        wm, bias = wb
        aq = _edge_agg(src3d, dst3d, *carry)
        nq = _tc_layer(aq, carry, s, wm, bias)
        return tuple(nq), None

    q3, _ = lax.scan(_layer_step, (q0a, q0b, zq, zq), (wstack, bstack))

    out = _tc_readout(q3, s,
                      batch_idx.astype(jnp.int32).reshape(NB, 1, BLK),
                      lin_W, lin_b)
    return out


# speedup vs baseline: 10.5690x; 1.0069x over previous
import jax
import jax.numpy as jnp
from jax import lax
from jax.experimental import pallas as pl
from jax.experimental.pallas import tpu as pltpu
from jax.experimental.pallas import tpu_sc as plsc

N = 10000
E = 320000
NF = 9
VOCAB = 119
EMB = 128
HID = 256
NG = 64

NC = 2
NS = 16
NW = NC * NS

_mesh = plsc.VectorSubcoreMesh(core_axis_name="c", subcore_axis_name="s")


KN = 80
NCHUNK = N // KN


def _sc_embed_deg_body(xg_hbm, emb_hbm, dst3d_hbm, h0_hbm, degp_hbm,
                       idxb, gA, gB, abuf, oneb, db2, zb, deg_sh,
                       sem_a, sg0, sg1, sd0, sd1):
    c = lax.axis_index("c")
    s = lax.axis_index("s")
    w = c * NS + s
    for r in range(40):
        zb[pl.ds(r * 16, 16)] = jnp.zeros((16,), jnp.float32)
    r0 = s * 640

    @pl.when(s < 15)
    def _():
        pltpu.sync_copy(zb, deg_sh.at[pl.ds(r0, 640)])

    @pl.when(s == 15)
    def _():
        pltpu.sync_copy(zb.at[pl.ds(0, 400)], deg_sh.at[pl.ds(9600, 400)])

    for r in range(8):
        oneb[pl.ds(r * 16, 16)] = jnp.ones((16,), jnp.float32)

    pltpu.sync_copy(dst3d_hbm.at[s], db2)
    plsc.subcore_barrier()

    kbase = c * 79
    ndeg = 79 - c

    def deg_fire(j, carry):
        pltpu.async_copy(oneb, deg_sh.at[db2.at[kbase + j]], sd0, add=True)
        return carry

    lax.fori_loop(0, ndeg, deg_fire, 0)

    rings = (gA, gB)
    rsems = (sg0, sg1)
    for i in range((NCHUNK + NW - 1) // NW):
        cid = w + i * NW

        @pl.when(cid < NCHUNK)
        def _(cid=cid):
            pltpu.sync_copy(xg_hbm.at[cid], idxb)
            da = pltpu.async_copy(emb_hbm.at[idxb.at[0]], abuf, sem_a)
            dg = pltpu.async_copy(emb_hbm.at[idxb.at[1]], rings[0], rsems[0])
            da.wait()
            for f in range(1, NF):
                dg.wait()
                rb = rings[(f - 1) % 2]
                if f + 1 < NF:
                    dg = pltpu.async_copy(emb_hbm.at[idxb.at[f + 1]],
                                          rings[f % 2], rsems[f % 2])

                def addrow(r, carry, rb=rb):
                    for rr in range(4):
                        for cc in range(EMB // 16):
                            plsc.addupdate(
                                abuf.at[4 * r + rr, pl.ds(cc * 16, 16)],
                                rb[4 * r + rr, pl.ds(cc * 16, 16)])
                    return carry

                lax.fori_loop(0, KN // 4, addrow, 0)
            pltpu.sync_copy(abuf, h0_hbm.at[pl.ds(cid * KN, KN)])

    def deg_drain(j, carry):
        pltpu.make_async_copy(oneb, deg_sh.at[db2.at[kbase]], sd0).wait()
        return carry

    lax.fori_loop(0, ndeg, deg_drain, 0)
    plsc.subcore_barrier()

    @pl.when(s < 15)
    def _():
        pltpu.sync_copy(deg_sh.at[pl.ds(r0, 640)], zb)
        pltpu.sync_copy(zb, degp_hbm.at[pl.ds(c * N + r0, 640)])

    @pl.when(s == 15)
    def _():
        pltpu.sync_copy(deg_sh.at[pl.ds(9600, 400)], zb.at[pl.ds(0, 400)])
        pltpu.sync_copy(zb.at[pl.ds(0, 400)],
                        degp_hbm.at[pl.ds(c * N + 9600, 400)])


_sc_embed_deg = pl.kernel(
    _sc_embed_deg_body,
    out_type=[jax.ShapeDtypeStruct((N, EMB), jnp.float32),
              jax.ShapeDtypeStruct((2 * N,), jnp.float32)],
    mesh=_mesh,
    scratch_types=[pltpu.VMEM((NF, KN), jnp.int32),
                   pltpu.VMEM((KN, EMB), jnp.float32),
                   pltpu.VMEM((KN, EMB), jnp.float32),
                   pltpu.VMEM((KN, EMB), jnp.float32),
                   pltpu.VMEM((128,), jnp.float32),
                   pltpu.VMEM((157, 128), jnp.int32),
                   pltpu.VMEM((640,), jnp.float32),
                   pltpu.VMEM_SHARED((N + 16,), jnp.float32),
                   pltpu.SemaphoreType.DMA,
                   pltpu.SemaphoreType.DMA,
                   pltpu.SemaphoreType.DMA,
                   pltpu.SemaphoreType.DMA,
                   pltpu.SemaphoreType.DMA],
    compiler_params=pltpu.CompilerParams(use_tc_tiling_on_sc=False),
)


QW = 64
KE = 128
EPT = E // NS
NCH = (EPT + KE - 1) // KE
EPAD = NCH * KE - EPT
NBUF = 4
NGRP = NCH // NBUF
DUMP = N


def _zero_g0(g0):
    def zrow(r, carry):
        for cc in range(QW // 16):
            g0[r, pl.ds(cc * 16, 16)] = jnp.zeros((16,), jnp.float32)
        return carry

    lax.fori_loop(0, KE, zrow, 0)


def _acc_rows(s, fn):
    r0 = s * 640

    @pl.when(s < 15)
    def _():
        for i in range(5):
            fn(r0 + i * KE, KE)

    @pl.when(s == 15)
    def _():
        for i in range(3):
            fn(9600 + i * KE, KE)
        fn(9984, 16)


def _agg_pass(q_hbm, out_hbm, sb2, db2, gbufs, gsems, ssems, acc_sh, s):
    g0 = gbufs[0]
    _zero_g0(g0)
    _acc_rows(s, lambda r, n: pltpu.sync_copy(
        g0.at[pl.ds(0, n)], acc_sh.at[pl.ds(r, n)]))
    plsc.subcore_barrier()

    for b in range(NBUF):
        pltpu.async_copy(q_hbm.at[sb2.at[b]], gbufs[b], gsems[b])

    def group(j, carry):
        descs = []
        for b in range(NBUF):
            k = NBUF * j + b
            pltpu.make_async_copy(q_hbm.at[sb2.at[0]], gbufs[b],
                                  gsems[b]).wait()
            descs.append(pltpu.async_copy(
                gbufs[b], acc_sh.at[db2.at[k]], ssems[b], add=True))
        for b in range(NBUF):
            k = NBUF * j + b
            descs[b].wait()

            @pl.when(k + NBUF < NCH)
            def _(b=b, k=k):
                pltpu.async_copy(q_hbm.at[sb2.at[k + NBUF]], gbufs[b],
                                 gsems[b])
        return carry

    lax.fori_loop(0, NGRP, group, 0)
    pltpu.make_async_copy(q_hbm.at[sb2.at[0]], gbufs[0], gsems[0]).wait()
    pltpu.async_copy(gbufs[0], acc_sh.at[db2.at[NCH - 1]], ssems[0],
                     add=True).wait()
    plsc.subcore_barrier()

    def wb(r, n):
        pltpu.sync_copy(acc_sh.at[pl.ds(r, n)], g0.at[pl.ds(0, n)])
        pltpu.sync_copy(g0.at[pl.ds(0, n)], out_hbm.at[pl.ds(r, n)])

    _acc_rows(s, wb)


def _edge_body(src3d_hbm, dst3d_hbm, q0_hbm, q1_hbm, q2_hbm, q3_hbm,
               a0_hbm, a1_hbm, a2_hbm, a3_hbm,
               sb2, db2, g0, g1, g2, g3, acc_sh,
               gs0, gs1, gs2, gs3, ss0, ss1, ss2, ss3):
    c = lax.axis_index("c")
    s = lax.axis_index("s")
    gbufs = (g0, g1, g2, g3)
    gsems = (gs0, gs1, gs2, gs3)
    ssems = (ss0, ss1, ss2, ss3)
    pltpu.sync_copy(src3d_hbm.at[s], sb2)
    pltpu.sync_copy(dst3d_hbm.at[s], db2)

    @pl.when(c == 0)
    def _():
        _agg_pass(q0_hbm, a0_hbm, sb2, db2, gbufs, gsems, ssems, acc_sh, s)
        plsc.subcore_barrier()
        _agg_pass(q1_hbm, a1_hbm, sb2, db2, gbufs, gsems, ssems, acc_sh, s)

    @pl.when(c == 1)
    def _():
        _agg_pass(q2_hbm, a2_hbm, sb2, db2, gbufs, gsems, ssems, acc_sh, s)
        plsc.subcore_barrier()
        _agg_pass(q3_hbm, a3_hbm, sb2, db2, gbufs, gsems, ssems, acc_sh, s)


_edge_agg = pl.kernel(
    _edge_body,
    out_type=[jax.ShapeDtypeStruct((N, QW), jnp.float32)] * 4,
    mesh=_mesh,
    scratch_types=[pltpu.VMEM((NCH, KE), jnp.int32),
                   pltpu.VMEM((NCH, KE), jnp.int32),
                   pltpu.VMEM((KE, QW), jnp.float32),
                   pltpu.VMEM((KE, QW), jnp.float32),
                   pltpu.VMEM((KE, QW), jnp.float32),
                   pltpu.VMEM((KE, QW), jnp.float32),
                   pltpu.VMEM_SHARED((N + 16, QW), jnp.float32),
                   pltpu.SemaphoreType.DMA,
                   pltpu.SemaphoreType.DMA,
                   pltpu.SemaphoreType.DMA,
                   pltpu.SemaphoreType.DMA,
                   pltpu.SemaphoreType.DMA,
                   pltpu.SemaphoreType.DMA,
                   pltpu.SemaphoreType.DMA,
                   pltpu.SemaphoreType.DMA],
    compiler_params=pltpu.CompilerParams(use_tc_tiling_on_sc=False),
)


BLK = 1000
NB = N // BLK


def _prescale_body(h0_ref, dpT_ref, s_ref, qa_ref, qb_ref):
    d = dpT_ref[:, 0:1] + dpT_ref[:, 1:2]
    s = lax.rsqrt(1.0 + d)
    s_ref[...] = s
    q = h0_ref[...] * s
    qa_ref[...] = q[:, :QW]
    qb_ref[...] = q[:, QW:]


def _tc_prescale(h0, dpT):
    return pl.pallas_call(
        _prescale_body,
        grid=(NB,),
        in_specs=[pl.BlockSpec((BLK, EMB), lambda i: (i, 0)),
                  pl.BlockSpec((BLK, 2), lambda i: (i, 0))],
        out_specs=[pl.BlockSpec((BLK, 1), lambda i: (i, 0)),
                   pl.BlockSpec((BLK, QW), lambda i: (i, 0)),
                   pl.BlockSpec((BLK, QW), lambda i: (i, 0))],
        out_shape=[jax.ShapeDtypeStruct((N, 1), jnp.float32),
                   jax.ShapeDtypeStruct((N, QW), jnp.float32),
                   jax.ShapeDtypeStruct((N, QW), jnp.float32)],
    )(h0, dpT)


def _layer_body(a0, a1, a2, a3, q0, q1, q2, q3, s_ref, w_ref, b_ref,
                o0, o1, o2, o3):
    s = s_ref[...]
    z = jnp.concatenate([a0[...] + q0[...], a1[...] + q1[...],
                         a2[...] + q2[...], a3[...] + q3[...]], axis=1) * s
    h = jnp.maximum(jnp.dot(z, w_ref[...]) + b_ref[...], 0.0)
    qn = h * s
    o0[...] = qn[:, 0 * QW:1 * QW]
    o1[...] = qn[:, 1 * QW:2 * QW]
    o2[...] = qn[:, 2 * QW:3 * QW]
    o3[...] = qn[:, 3 * QW:4 * QW]


_qspec = pl.BlockSpec((BLK, QW), lambda i: (i, 0))


def _tc_layer(aq, qq, s, wm, bias):
    return pl.pallas_call(
        _layer_body,
        grid=(NB,),
        in_specs=[_qspec] * 8 + [
            pl.BlockSpec((BLK, 1), lambda i: (i, 0)),
            pl.BlockSpec((HID, HID), lambda i: (0, 0)),
            pl.BlockSpec((1, HID), lambda i: (0, 0))],
        out_specs=[_qspec] * 4,
        out_shape=[jax.ShapeDtypeStruct((N, QW), jnp.float32)] * 4,
    )(*aq, *qq, s, wm, bias.reshape(1, HID))


def _readout_body(q0, q1, q2, q3, s_ref, bidx_ref, linw_ref, linb_ref,
                  out_ref, sums_scr, cnts_scr):
    i = pl.program_id(0)
    s = s_ref[...]
    h3 = jnp.concatenate([q0[...], q1[...], q2[...], q3[...]], axis=1) / s
    gids = lax.broadcasted_iota(jnp.int32, (NG, BLK), 0)
    ohT = (bidx_ref[0] == gids).astype(jnp.float32)
    bs = jnp.dot(ohT, h3)
    bc = jnp.sum(ohT, axis=1, keepdims=True)

    @pl.when(i == 0)
    def _():
        sums_scr[...] = bs
        cnts_scr[...] = bc

    @pl.when(i > 0)
    def _():
        sums_scr[...] += bs
        cnts_scr[...] += bc

    @pl.when(i == NB - 1)
    def _():
        mean = sums_scr[...] / jnp.maximum(cnts_scr[...], 1.0)
        out_ref[...] = jax.nn.sigmoid(jnp.dot(mean, linw_ref[...])
                                      + linb_ref[...])


def _tc_readout(qq, s, bidx_3d, lin_W, lin_b):
    return pl.pallas_call(
        _readout_body,
        grid=(NB,),
        in_specs=[_qspec] * 4 + [
            pl.BlockSpec((BLK, 1), lambda i: (i, 0)),
            pl.BlockSpec((1, 1, BLK), lambda i: (i, 0, 0)),
            pl.BlockSpec((HID, 1), lambda i: (0, 0)),
            pl.BlockSpec((1, 1), lambda i: (0, 0))],
        out_specs=pl.BlockSpec((NG, 1), lambda i: (0, 0)),
        out_shape=jax.ShapeDtypeStruct((NG, 1), jnp.float32),
        scratch_shapes=[pltpu.VMEM((NG, HID), jnp.float32),
                        pltpu.VMEM((NG, 1), jnp.float32)],
    )(*qq, s, bidx_3d, lin_W, lin_b.reshape(1, 1))


def kernel(x, edge_index, batch_idx, atom_emb, W0, b0, W1, b1, W2, b2,
           lin_W, lin_b):
    x = x.astype(jnp.int32)
    src = edge_index[0].astype(jnp.int32)
    dst = edge_index[1].astype(jnp.int32)
    xg = jnp.transpose(
        x + (jnp.arange(NF, dtype=jnp.int32) * VOCAB)[None, :]
    ).reshape(NF, NCHUNK, KN).transpose(1, 0, 2)
    flat_emb = atom_emb.reshape(NF * VOCAB, EMB)

    srcr = src.reshape(NS, EPT)
    dstr = dst.reshape(NS, EPT)
    src3d = jnp.concatenate(
        [srcr, jnp.zeros((NS, EPAD), jnp.int32)], axis=1).reshape(NS, NCH, KE)
    dst3d = jnp.concatenate(
        [dstr, jnp.full((NS, EPAD), DUMP, jnp.int32)],
        axis=1).reshape(NS, NCH, KE)

    h0, degp = _sc_embed_deg(xg, flat_emb, dst3d)
    s, q0a, q0b = _tc_prescale(h0, jnp.transpose(degp.reshape(2, N)))

    w0p = jnp.concatenate([W0, jnp.zeros((HID - EMB, HID), jnp.float32)], 0)
    wstack = jnp.stack([w0p, W1, W2])
    bstack = jnp.stack([b0, b1, b2])
    zq = jnp.zeros((N, QW), jnp.float32)

    def _layer_step(carry, wb):
        wm, bias = wb
        aq = _edge_agg(src3d, dst3d, *carry)
        nq = _tc_layer(aq, carry, s, wm, bias)
        return tuple(nq), None

    q3, _ = lax.scan(_layer_step, (q0a, q0b, zq, zq), (wstack, bstack))

    out = _tc_readout(q3, s,
                      batch_idx.astype(jnp.int32).reshape(NB, 1, BLK),
                      lin_W, lin_b)
    return out


# speedup vs baseline: 14.0609x; 1.3304x over previous
import jax
import jax.numpy as jnp
from jax import lax
from jax.experimental import pallas as pl
from jax.experimental.pallas import tpu as pltpu
from jax.experimental.pallas import tpu_sc as plsc

N = 10000
E = 320000
NF = 9
VOCAB = 119
EMB = 128
HID = 256
NG = 64

NC = 2
NS = 16
NW = NC * NS

_mesh = plsc.VectorSubcoreMesh(core_axis_name="c", subcore_axis_name="s")


def _sc_deg_body(dst3d_hbm, degp_hbm, oneb, db2, zb, deg_sh, sd0):
    c = lax.axis_index("c")
    s = lax.axis_index("s")
    for r in range(40):
        zb[pl.ds(r * 16, 16)] = jnp.zeros((16,), jnp.float32)
    r0 = s * 640

    @pl.when(s < 15)
    def _():
        pltpu.sync_copy(zb, deg_sh.at[pl.ds(r0, 640)])

    @pl.when(s == 15)
    def _():
        pltpu.sync_copy(zb.at[pl.ds(0, 400)], deg_sh.at[pl.ds(9600, 400)])

    for r in range(8):
        oneb[pl.ds(r * 16, 16)] = jnp.ones((16,), jnp.float32)

    pltpu.sync_copy(dst3d_hbm.at[s], db2)
    plsc.subcore_barrier()

    kbase = c * 79
    ndeg = 79 - c

    def deg_fire(j, carry):
        pltpu.async_copy(oneb, deg_sh.at[db2.at[kbase + j]], sd0, add=True)
        return carry

    lax.fori_loop(0, ndeg, deg_fire, 0)

    def deg_drain(j, carry):
        pltpu.make_async_copy(oneb, deg_sh.at[db2.at[kbase]], sd0).wait()
        return carry

    lax.fori_loop(0, ndeg, deg_drain, 0)
    plsc.subcore_barrier()

    @pl.when(s < 15)
    def _():
        pltpu.sync_copy(deg_sh.at[pl.ds(r0, 640)], zb)
        pltpu.sync_copy(zb, degp_hbm.at[pl.ds(c * N + r0, 640)])

    @pl.when(s == 15)
    def _():
        pltpu.sync_copy(deg_sh.at[pl.ds(9600, 400)], zb.at[pl.ds(0, 400)])
        pltpu.sync_copy(zb.at[pl.ds(0, 400)],
                        degp_hbm.at[pl.ds(c * N + 9600, 400)])


_sc_deg = pl.kernel(
    _sc_deg_body,
    out_type=jax.ShapeDtypeStruct((2 * N,), jnp.float32),
    mesh=_mesh,
    scratch_types=[pltpu.VMEM((128,), jnp.float32),
                   pltpu.VMEM((157, 128), jnp.int32),
                   pltpu.VMEM((640,), jnp.float32),
                   pltpu.VMEM_SHARED((N + 16,), jnp.float32),
                   pltpu.SemaphoreType.DMA],
    compiler_params=pltpu.CompilerParams(use_tc_tiling_on_sc=False),
)


QW = 64
KE = 128
EPT = E // NS
NCH = (EPT + KE - 1) // KE
EPAD = NCH * KE - EPT
NBUF = 4
NGRP = NCH // NBUF
DUMP = N


def _zero_g0(g0):
    def zrow(r, carry):
        for cc in range(QW // 16):
            g0[r, pl.ds(cc * 16, 16)] = jnp.zeros((16,), jnp.float32)
        return carry

    lax.fori_loop(0, KE, zrow, 0)


def _acc_rows(s, fn):
    r0 = s * 640

    @pl.when(s < 15)
    def _():
        for i in range(5):
            fn(r0 + i * KE, KE)

    @pl.when(s == 15)
    def _():
        for i in range(3):
            fn(9600 + i * KE, KE)
        fn(9984, 16)


def _agg_pass(q_hbm, out_hbm, sb2, db2, gbufs, gsems, ssems, acc_sh, s):
    g0 = gbufs[0]
    _zero_g0(g0)
    _acc_rows(s, lambda r, n: pltpu.sync_copy(
        g0.at[pl.ds(0, n)], acc_sh.at[pl.ds(r, n)]))
    plsc.subcore_barrier()

    for b in range(NBUF):
        pltpu.async_copy(q_hbm.at[sb2.at[b]], gbufs[b], gsems[b])

    def group(j, carry):
        descs = []
        for b in range(NBUF):
            k = NBUF * j + b
            pltpu.make_async_copy(q_hbm.at[sb2.at[0]], gbufs[b],
                                  gsems[b]).wait()
            descs.append(pltpu.async_copy(
                gbufs[b], acc_sh.at[db2.at[k]], ssems[b], add=True))
        for b in range(NBUF):
            k = NBUF * j + b
            descs[b].wait()

            @pl.when(k + NBUF < NCH)
            def _(b=b, k=k):
                pltpu.async_copy(q_hbm.at[sb2.at[k + NBUF]], gbufs[b],
                                 gsems[b])
        return carry

    lax.fori_loop(0, NGRP, group, 0)
    pltpu.make_async_copy(q_hbm.at[sb2.at[0]], gbufs[0], gsems[0]).wait()
    pltpu.async_copy(gbufs[0], acc_sh.at[db2.at[NCH - 1]], ssems[0],
                     add=True).wait()
    plsc.subcore_barrier()

    def wb(r, n):
        pltpu.sync_copy(acc_sh.at[pl.ds(r, n)], g0.at[pl.ds(0, n)])
        pltpu.sync_copy(g0.at[pl.ds(0, n)], out_hbm.at[pl.ds(r, n)])

    _acc_rows(s, wb)


def _edge_body(src3d_hbm, dst3d_hbm, q0_hbm, q1_hbm, q2_hbm, q3_hbm,
               a0_hbm, a1_hbm, a2_hbm, a3_hbm,
               sb2, db2, g0, g1, g2, g3, acc_sh,
               gs0, gs1, gs2, gs3, ss0, ss1, ss2, ss3):
    c = lax.axis_index("c")
    s = lax.axis_index("s")
    gbufs = (g0, g1, g2, g3)
    gsems = (gs0, gs1, gs2, gs3)
    ssems = (ss0, ss1, ss2, ss3)
    pltpu.sync_copy(src3d_hbm.at[s], sb2)
    pltpu.sync_copy(dst3d_hbm.at[s], db2)

    @pl.when(c == 0)
    def _():
        _agg_pass(q0_hbm, a0_hbm, sb2, db2, gbufs, gsems, ssems, acc_sh, s)
        plsc.subcore_barrier()
        _agg_pass(q1_hbm, a1_hbm, sb2, db2, gbufs, gsems, ssems, acc_sh, s)

    @pl.when(c == 1)
    def _():
        _agg_pass(q2_hbm, a2_hbm, sb2, db2, gbufs, gsems, ssems, acc_sh, s)
        plsc.subcore_barrier()
        _agg_pass(q3_hbm, a3_hbm, sb2, db2, gbufs, gsems, ssems, acc_sh, s)


_edge_agg = pl.kernel(
    _edge_body,
    out_type=[jax.ShapeDtypeStruct((N, QW), jnp.float32)] * 4,
    mesh=_mesh,
    scratch_types=[pltpu.VMEM((NCH, KE), jnp.int32),
                   pltpu.VMEM((NCH, KE), jnp.int32),
                   pltpu.VMEM((KE, QW), jnp.float32),
                   pltpu.VMEM((KE, QW), jnp.float32),
                   pltpu.VMEM((KE, QW), jnp.float32),
                   pltpu.VMEM((KE, QW), jnp.float32),
                   pltpu.VMEM_SHARED((N + 16, QW), jnp.float32),
                   pltpu.SemaphoreType.DMA,
                   pltpu.SemaphoreType.DMA,
                   pltpu.SemaphoreType.DMA,
                   pltpu.SemaphoreType.DMA,
                   pltpu.SemaphoreType.DMA,
                   pltpu.SemaphoreType.DMA,
                   pltpu.SemaphoreType.DMA,
                   pltpu.SemaphoreType.DMA],
    compiler_params=pltpu.CompilerParams(use_tc_tiling_on_sc=False),
)


BLK = 1000
NB = N // BLK


def _prescale_body(x_ref, emb_ref, dpT_ref, s_ref, qa_ref, qb_ref):
    vids = lax.broadcasted_iota(jnp.int32, (BLK, 128), 1)
    h = jnp.zeros((BLK, EMB), jnp.float32)
    for f in range(NF):
        oh = (x_ref[:, f:f + 1] == vids).astype(jnp.float32)
        h = h + jnp.dot(oh, emb_ref[f])
    d = dpT_ref[:, 0:1] + dpT_ref[:, 1:2]
    s = lax.rsqrt(1.0 + d)
    s_ref[...] = s
    q = h * s
    qa_ref[...] = q[:, :QW]
    qb_ref[...] = q[:, QW:]


def _tc_prescale(x, emb_pad, dpT):
    return pl.pallas_call(
        _prescale_body,
        grid=(NB,),
        in_specs=[pl.BlockSpec((BLK, NF), lambda i: (i, 0)),
                  pl.BlockSpec((NF, 128, EMB), lambda i: (0, 0, 0)),
                  pl.BlockSpec((BLK, 2), lambda i: (i, 0))],
        out_specs=[pl.BlockSpec((BLK, 1), lambda i: (i, 0)),
                   pl.BlockSpec((BLK, QW), lambda i: (i, 0)),
                   pl.BlockSpec((BLK, QW), lambda i: (i, 0))],
        out_shape=[jax.ShapeDtypeStruct((N, 1), jnp.float32),
                   jax.ShapeDtypeStruct((N, QW), jnp.float32),
                   jax.ShapeDtypeStruct((N, QW), jnp.float32)],
    )(x, emb_pad, dpT)


def _layer_body(a0, a1, a2, a3, q0, q1, q2, q3, s_ref, w_ref, b_ref,
                o0, o1, o2, o3):
    s = s_ref[...]
    z = jnp.concatenate([a0[...] + q0[...], a1[...] + q1[...],
                         a2[...] + q2[...], a3[...] + q3[...]], axis=1) * s
    h = jnp.maximum(jnp.dot(z, w_ref[...]) + b_ref[...], 0.0)
    qn = h * s
    o0[...] = qn[:, 0 * QW:1 * QW]
    o1[...] = qn[:, 1 * QW:2 * QW]
    o2[...] = qn[:, 2 * QW:3 * QW]
    o3[...] = qn[:, 3 * QW:4 * QW]


_qspec = pl.BlockSpec((BLK, QW), lambda i: (i, 0))


def _tc_layer(aq, qq, s, wm, bias):
    return pl.pallas_call(
        _layer_body,
        grid=(NB,),
        in_specs=[_qspec] * 8 + [
            pl.BlockSpec((BLK, 1), lambda i: (i, 0)),
            pl.BlockSpec((HID, HID), lambda i: (0, 0)),
            pl.BlockSpec((1, HID), lambda i: (0, 0))],
        out_specs=[_qspec] * 4,
        out_shape=[jax.ShapeDtypeStruct((N, QW), jnp.float32)] * 4,
    )(*aq, *qq, s, wm, bias.reshape(1, HID))


def _readout_body(q0, q1, q2, q3, s_ref, bidx_ref, linw_ref, linb_ref,
                  out_ref, sums_scr, cnts_scr):
    i = pl.program_id(0)
    s = s_ref[...]
    h3 = jnp.concatenate([q0[...], q1[...], q2[...], q3[...]], axis=1) / s
    gids = lax.broadcasted_iota(jnp.int32, (NG, BLK), 0)
    ohT = (bidx_ref[0] == gids).astype(jnp.float32)
    bs = jnp.dot(ohT, h3)
    bc = jnp.sum(ohT, axis=1, keepdims=True)

    @pl.when(i == 0)
    def _():
        sums_scr[...] = bs
        cnts_scr[...] = bc

    @pl.when(i > 0)
    def _():
        sums_scr[...] += bs
        cnts_scr[...] += bc

    @pl.when(i == NB - 1)
    def _():
        mean = sums_scr[...] / jnp.maximum(cnts_scr[...], 1.0)
        out_ref[...] = jax.nn.sigmoid(jnp.dot(mean, linw_ref[...])
                                      + linb_ref[...])


def _tc_readout(qq, s, bidx_3d, lin_W, lin_b):
    return pl.pallas_call(
        _readout_body,
        grid=(NB,),
        in_specs=[_qspec] * 4 + [
            pl.BlockSpec((BLK, 1), lambda i: (i, 0)),
            pl.BlockSpec((1, 1, BLK), lambda i: (i, 0, 0)),
            pl.BlockSpec((HID, 1), lambda i: (0, 0)),
            pl.BlockSpec((1, 1), lambda i: (0, 0))],
        out_specs=pl.BlockSpec((NG, 1), lambda i: (0, 0)),
        out_shape=jax.ShapeDtypeStruct((NG, 1), jnp.float32),
        scratch_shapes=[pltpu.VMEM((NG, HID), jnp.float32),
                        pltpu.VMEM((NG, 1), jnp.float32)],
    )(*qq, s, bidx_3d, lin_W, lin_b.reshape(1, 1))


def kernel(x, edge_index, batch_idx, atom_emb, W0, b0, W1, b1, W2, b2,
           lin_W, lin_b):
    x = x.astype(jnp.int32)
    src = edge_index[0].astype(jnp.int32)
    dst = edge_index[1].astype(jnp.int32)
    emb_pad = jnp.pad(atom_emb, ((0, 0), (0, 128 - VOCAB), (0, 0)))

    srcr = src.reshape(NS, EPT)
    dstr = dst.reshape(NS, EPT)
    src3d = jnp.concatenate(
        [srcr, jnp.zeros((NS, EPAD), jnp.int32)], axis=1).reshape(NS, NCH, KE)
    dst3d = jnp.concatenate(
        [dstr, jnp.full((NS, EPAD), DUMP, jnp.int32)],
        axis=1).reshape(NS, NCH, KE)

    degp = _sc_deg(dst3d)
    s, q0a, q0b = _tc_prescale(x, emb_pad, jnp.transpose(degp.reshape(2, N)))

    w0p = jnp.concatenate([W0, jnp.zeros((HID - EMB, HID), jnp.float32)], 0)
    wstack = jnp.stack([w0p, W1, W2])
    bstack = jnp.stack([b0, b1, b2])
    zq = jnp.zeros((N, QW), jnp.float32)

    def _layer_step(carry, wb):
        wm, bias = wb
        aq = _edge_agg(src3d, dst3d, *carry)
        nq = _tc_layer(aq, carry, s, wm, bias)
        return tuple(nq), None

    q3, _ = lax.scan(_layer_step, (q0a, q0b, zq, zq), (wstack, bstack))

    out = _tc_readout(q3, s,
                      batch_idx.astype(jnp.int32).reshape(NB, 1, BLK),
                      lin_W, lin_b)
    return out


# speedup vs baseline: 14.8490x; 1.0561x over previous
import jax
import jax.numpy as jnp
from jax import lax
from jax.experimental import pallas as pl
from jax.experimental.pallas import tpu as pltpu
from jax.experimental.pallas import tpu_sc as plsc

N = 10000
E = 320000
NF = 9
VOCAB = 119
EMB = 128
HID = 256
NG = 64

NC = 2
NS = 16
NW = NC * NS

_mesh = plsc.VectorSubcoreMesh(core_axis_name="c", subcore_axis_name="s")


def _sc_deg_body(dst3d_hbm, degp_hbm, oneb, db2, zb, deg_sh, sd0):
    c = lax.axis_index("c")
    s = lax.axis_index("s")
    for r in range(40):
        zb[pl.ds(r * 16, 16)] = jnp.zeros((16,), jnp.float32)
    r0 = s * 640

    @pl.when(s < 15)
    def _():
        pltpu.sync_copy(zb, deg_sh.at[pl.ds(r0, 640)])

    @pl.when(s == 15)
    def _():
        pltpu.sync_copy(zb.at[pl.ds(0, 400)], deg_sh.at[pl.ds(9600, 400)])

    for r in range(8):
        oneb[pl.ds(r * 16, 16)] = jnp.ones((16,), jnp.float32)

    pltpu.sync_copy(dst3d_hbm.at[s], db2)
    plsc.subcore_barrier()

    kbase = c * 79
    ndeg = 79 - c

    def deg_fire(j, carry):
        pltpu.async_copy(oneb, deg_sh.at[db2.at[kbase + j]], sd0, add=True)
        return carry

    lax.fori_loop(0, ndeg, deg_fire, 0)

    def deg_drain(j, carry):
        pltpu.make_async_copy(oneb, deg_sh.at[db2.at[kbase]], sd0).wait()
        return carry

    lax.fori_loop(0, ndeg, deg_drain, 0)
    plsc.subcore_barrier()

    @pl.when(s < 15)
    def _():
        pltpu.sync_copy(deg_sh.at[pl.ds(r0, 640)], zb)
        pltpu.sync_copy(zb, degp_hbm.at[pl.ds(c * N + r0, 640)])

    @pl.when(s == 15)
    def _():
        pltpu.sync_copy(deg_sh.at[pl.ds(9600, 400)], zb.at[pl.ds(0, 400)])
        pltpu.sync_copy(zb.at[pl.ds(0, 400)],
                        degp_hbm.at[pl.ds(c * N + 9600, 400)])


_sc_deg = pl.kernel(
    _sc_deg_body,
    out_type=jax.ShapeDtypeStruct((2 * N,), jnp.float32),
    mesh=_mesh,
    scratch_types=[pltpu.VMEM((128,), jnp.float32),
                   pltpu.VMEM((157, 128), jnp.int32),
                   pltpu.VMEM((640,), jnp.float32),
                   pltpu.VMEM_SHARED((N + 16,), jnp.float32),
                   pltpu.SemaphoreType.DMA],
    compiler_params=pltpu.CompilerParams(use_tc_tiling_on_sc=False),
)


QW = 64
KE = 128
EPT = E // NS
NCH = (EPT + KE - 1) // KE
EPAD = NCH * KE - EPT
NBUF = 4
NGRP = NCH // NBUF
DUMP = N


def _zero_g0(g0):
    def zrow(r, carry):
        for cc in range(QW // 16):
            g0[r, pl.ds(cc * 16, 16)] = jnp.zeros((16,), jnp.float32)
        return carry

    lax.fori_loop(0, KE, zrow, 0)


def _acc_rows(s, fn):
    r0 = s * 640

    @pl.when(s < 15)
    def _():
        for i in range(5):
            fn(r0 + i * KE, KE)

    @pl.when(s == 15)
    def _():
        for i in range(3):
            fn(9600 + i * KE, KE)
        fn(9984, 16)


def _agg_pass(q_hbm, out_hbm, sb2, db2, gbufs, gsems, ssems, acc_sh, s):
    g0 = gbufs[0]
    _zero_g0(g0)
    _acc_rows(s, lambda r, n: pltpu.sync_copy(
        g0.at[pl.ds(0, n)], acc_sh.at[pl.ds(r, n)]))
    plsc.subcore_barrier()

    for b in range(NBUF):
        pltpu.async_copy(q_hbm.at[sb2.at[b]], gbufs[b], gsems[b])

    def group(j, carry):
        descs = []
        for b in range(NBUF):
            k = NBUF * j + b
            pltpu.make_async_copy(q_hbm.at[sb2.at[0]], gbufs[b],
                                  gsems[b]).wait()
            descs.append(pltpu.async_copy(
                gbufs[b], acc_sh.at[db2.at[k]], ssems[b], add=True))
        for b in range(NBUF):
            k = NBUF * j + b
            descs[b].wait()

            @pl.when(k + NBUF < NCH)
            def _(b=b, k=k):
                pltpu.async_copy(q_hbm.at[sb2.at[k + NBUF]], gbufs[b],
                                 gsems[b])
        return carry

    lax.fori_loop(0, NGRP, group, 0)
    pltpu.make_async_copy(q_hbm.at[sb2.at[0]], gbufs[0], gsems[0]).wait()
    pltpu.async_copy(gbufs[0], acc_sh.at[db2.at[NCH - 1]], ssems[0],
                     add=True).wait()
    plsc.subcore_barrier()

    def wb(r, n):
        pltpu.sync_copy(acc_sh.at[pl.ds(r, n)], g0.at[pl.ds(0, n)])
        pltpu.sync_copy(g0.at[pl.ds(0, n)], out_hbm.at[pl.ds(r, n)])

    _acc_rows(s, wb)


def _edge_body(src3d_hbm, dst3d_hbm, q0_hbm, q1_hbm, q2_hbm, q3_hbm,
               a0_hbm, a1_hbm, a2_hbm, a3_hbm,
               sb2, db2, g0, g1, g2, g3, acc_sh,
               gs0, gs1, gs2, gs3, ss0, ss1, ss2, ss3):
    c = lax.axis_index("c")
    s = lax.axis_index("s")
    gbufs = (g0, g1, g2, g3)
    gsems = (gs0, gs1, gs2, gs3)
    ssems = (ss0, ss1, ss2, ss3)
    pltpu.sync_copy(src3d_hbm.at[s], sb2)
    pltpu.sync_copy(dst3d_hbm.at[s], db2)

    @pl.when(c == 0)
    def _():
        _agg_pass(q0_hbm, a0_hbm, sb2, db2, gbufs, gsems, ssems, acc_sh, s)
        plsc.subcore_barrier()
        _agg_pass(q1_hbm, a1_hbm, sb2, db2, gbufs, gsems, ssems, acc_sh, s)

    @pl.when(c == 1)
    def _():
        _agg_pass(q2_hbm, a2_hbm, sb2, db2, gbufs, gsems, ssems, acc_sh, s)
        plsc.subcore_barrier()
        _agg_pass(q3_hbm, a3_hbm, sb2, db2, gbufs, gsems, ssems, acc_sh, s)


_edge_agg = pl.kernel(
    _edge_body,
    out_type=[jax.ShapeDtypeStruct((N, QW), jnp.float32)] * 4,
    mesh=_mesh,
    scratch_types=[pltpu.VMEM((NCH, KE), jnp.int32),
                   pltpu.VMEM((NCH, KE), jnp.int32),
                   pltpu.VMEM((KE, QW), jnp.float32),
                   pltpu.VMEM((KE, QW), jnp.float32),
                   pltpu.VMEM((KE, QW), jnp.float32),
                   pltpu.VMEM((KE, QW), jnp.float32),
                   pltpu.VMEM_SHARED((N + 16, QW), jnp.float32),
                   pltpu.SemaphoreType.DMA,
                   pltpu.SemaphoreType.DMA,
                   pltpu.SemaphoreType.DMA,
                   pltpu.SemaphoreType.DMA,
                   pltpu.SemaphoreType.DMA,
                   pltpu.SemaphoreType.DMA,
                   pltpu.SemaphoreType.DMA,
                   pltpu.SemaphoreType.DMA],
    compiler_params=pltpu.CompilerParams(use_tc_tiling_on_sc=False),
)


BLK = 1000
NB = N // BLK


def _prescale_body(x_ref, emb_ref, dpT_ref, s_ref, qa_ref, qb_ref):
    vids = lax.broadcasted_iota(jnp.int32, (BLK, 128), 1)
    h = jnp.zeros((BLK, EMB), jnp.float32)
    for f in range(NF):
        oh = (x_ref[:, f:f + 1] == vids).astype(jnp.float32)
        h = h + jnp.dot(oh, emb_ref[f])
    d = dpT_ref[:, 0:1] + dpT_ref[:, 1:2]
    s = lax.rsqrt(1.0 + d)
    s_ref[...] = s
    q = h * s
    qa_ref[...] = q[:, :QW]
    qb_ref[...] = q[:, QW:]


def _tc_prescale(x, emb_pad, dpT):
    return pl.pallas_call(
        _prescale_body,
        grid=(NB,),
        in_specs=[pl.BlockSpec((BLK, NF), lambda i: (i, 0)),
                  pl.BlockSpec((NF, 128, EMB), lambda i: (0, 0, 0)),
                  pl.BlockSpec((BLK, 2), lambda i: (i, 0))],
        out_specs=[pl.BlockSpec((BLK, 1), lambda i: (i, 0)),
                   pl.BlockSpec((BLK, QW), lambda i: (i, 0)),
                   pl.BlockSpec((BLK, QW), lambda i: (i, 0))],
        out_shape=[jax.ShapeDtypeStruct((N, 1), jnp.float32),
                   jax.ShapeDtypeStruct((N, QW), jnp.float32),
                   jax.ShapeDtypeStruct((N, QW), jnp.float32)],
    )(x, emb_pad, dpT)


def _layer_body(a0, a1, a2, a3, q0, q1, q2, q3, s_ref, w_ref, b_ref,
                o0, o1, o2, o3):
    s = s_ref[...]
    z = jnp.concatenate([a0[...] + q0[...], a1[...] + q1[...],
                         a2[...] + q2[...], a3[...] + q3[...]], axis=1) * s
    h = jnp.maximum(jnp.dot(z, w_ref[...]) + b_ref[...], 0.0)
    qn = h * s
    o0[...] = qn[:, 0 * QW:1 * QW]
    o1[...] = qn[:, 1 * QW:2 * QW]
    o2[...] = qn[:, 2 * QW:3 * QW]
    o3[...] = qn[:, 3 * QW:4 * QW]


_qspec = pl.BlockSpec((BLK, QW), lambda i: (i, 0))


def _tc_layer(aq, qq, s, wm, bias):
    return pl.pallas_call(
        _layer_body,
        grid=(NB,),
        in_specs=[_qspec] * 8 + [
            pl.BlockSpec((BLK, 1), lambda i: (i, 0)),
            pl.BlockSpec((HID, HID), lambda i: (0, 0)),
            pl.BlockSpec((1, HID), lambda i: (0, 0))],
        out_specs=[_qspec] * 4,
        out_shape=[jax.ShapeDtypeStruct((N, QW), jnp.float32)] * 4,
    )(*aq, *qq, s, wm, bias.reshape(1, HID))


def _readout_body(q0, q1, q2, q3, s_ref, bidx_ref, linw_ref, linb_ref,
                  out_ref, sums_scr, cnts_scr):
    i = pl.program_id(0)
    s = s_ref[...]
    h3 = jnp.concatenate([q0[...], q1[...], q2[...], q3[...]], axis=1) / s
    gids = lax.broadcasted_iota(jnp.int32, (NG, BLK), 0)
    ohT = (bidx_ref[0] == gids).astype(jnp.float32)
    bs = jnp.dot(ohT, h3)
    bc = jnp.sum(ohT, axis=1, keepdims=True)

    @pl.when(i == 0)
    def _():
        sums_scr[...] = bs
        cnts_scr[...] = bc

    @pl.when(i > 0)
    def _():
        sums_scr[...] += bs
        cnts_scr[...] += bc

    @pl.when(i == NB - 1)
    def _():
        mean = sums_scr[...] / jnp.maximum(cnts_scr[...], 1.0)
        out_ref[...] = jax.nn.sigmoid(jnp.dot(mean, linw_ref[...])
                                      + linb_ref[...])


def _tc_readout(qq, s, bidx_3d, lin_W, lin_b):
    return pl.pallas_call(
        _readout_body,
        grid=(NB,),
        in_specs=[_qspec] * 4 + [
            pl.BlockSpec((BLK, 1), lambda i: (i, 0)),
            pl.BlockSpec((1, 1, BLK), lambda i: (i, 0, 0)),
            pl.BlockSpec((HID, 1), lambda i: (0, 0)),
            pl.BlockSpec((1, 1), lambda i: (0, 0))],
        out_specs=pl.BlockSpec((NG, 1), lambda i: (0, 0)),
        out_shape=jax.ShapeDtypeStruct((NG, 1), jnp.float32),
        scratch_shapes=[pltpu.VMEM((NG, HID), jnp.float32),
                        pltpu.VMEM((NG, 1), jnp.float32)],
    )(*qq, s, bidx_3d, lin_W, lin_b.reshape(1, 1))


def kernel(x, edge_index, batch_idx, atom_emb, W0, b0, W1, b1, W2, b2,
           lin_W, lin_b):
    x = x.astype(jnp.int32)
    src = edge_index[0].astype(jnp.int32)
    dst = edge_index[1].astype(jnp.int32)
    emb_pad = jnp.pad(atom_emb, ((0, 0), (0, 128 - VOCAB), (0, 0)))

    srcr = src.reshape(NS, EPT)
    dstr = dst.reshape(NS, EPT)
    src3d = jnp.concatenate(
        [srcr, jnp.zeros((NS, EPAD), jnp.int32)], axis=1).reshape(NS, NCH, KE)
    dst3d = jnp.concatenate(
        [dstr, jnp.full((NS, EPAD), DUMP, jnp.int32)],
        axis=1).reshape(NS, NCH, KE)

    degp = _sc_deg(dst3d)
    s, q0a, q0b = _tc_prescale(x, emb_pad, jnp.transpose(degp.reshape(2, N)))

    w0p = jnp.concatenate([W0, jnp.zeros((HID - EMB, HID), jnp.float32)], 0)
    wstack = jnp.stack([w0p, W1, W2])
    bstack = jnp.stack([b0, b1, b2])
    zq = jnp.zeros((N, QW), jnp.float32)

    def _layer_step(carry, wb):
        wm, bias = wb
        aq = _edge_agg(src3d, dst3d, *carry)
        nq = _tc_layer(aq, carry, s, wm, bias)
        return tuple(nq), None

    carry = (q0a, q0b, zq, zq)
    for li in range(3):
        carry, _ = _layer_step(carry, (wstack[li], bstack[li]))
    q3 = carry

    out = _tc_readout(q3, s,
                      batch_idx.astype(jnp.int32).reshape(NB, 1, BLK),
                      lin_W, lin_b)
    return out


# speedup vs baseline: 15.5664x; 1.0483x over previous
import jax
import jax.numpy as jnp
from jax import lax
from jax.experimental import pallas as pl
from jax.experimental.pallas import tpu as pltpu
from jax.experimental.pallas import tpu_sc as plsc

N = 10000
E = 320000
NF = 9
VOCAB = 119
EMB = 128
HID = 256
NG = 64

NC = 2
NS = 16
NW = NC * NS

_mesh = plsc.VectorSubcoreMesh(core_axis_name="c", subcore_axis_name="s")


def _sc_deg_body(dst3d_hbm, degp_hbm, oneb, db2, zb, deg_sh, sd0):
    c = lax.axis_index("c")
    s = lax.axis_index("s")
    for r in range(40):
        zb[pl.ds(r * 16, 16)] = jnp.zeros((16,), jnp.float32)
    r0 = s * 640

    @pl.when(s < 15)
    def _():
        pltpu.sync_copy(zb, deg_sh.at[pl.ds(r0, 640)])

    @pl.when(s == 15)
    def _():
        pltpu.sync_copy(zb.at[pl.ds(0, 400)], deg_sh.at[pl.ds(9600, 400)])

    for r in range(8):
        oneb[pl.ds(r * 16, 16)] = jnp.ones((16,), jnp.float32)

    pltpu.sync_copy(dst3d_hbm.at[s], db2)
    plsc.subcore_barrier()

    kbase = c * 79
    ndeg = 79 - c

    def deg_fire(j, carry):
        pltpu.async_copy(oneb, deg_sh.at[db2.at[kbase + j]], sd0, add=True)
        return carry

    lax.fori_loop(0, ndeg, deg_fire, 0)

    def deg_drain(j, carry):
        pltpu.make_async_copy(oneb, deg_sh.at[db2.at[kbase]], sd0).wait()
        return carry

    lax.fori_loop(0, ndeg, deg_drain, 0)
    plsc.subcore_barrier()

    @pl.when(s < 15)
    def _():
        pltpu.sync_copy(deg_sh.at[pl.ds(r0, 640)], zb)
        pltpu.sync_copy(zb, degp_hbm.at[pl.ds(c * N + r0, 640)])

    @pl.when(s == 15)
    def _():
        pltpu.sync_copy(deg_sh.at[pl.ds(9600, 400)], zb.at[pl.ds(0, 400)])
        pltpu.sync_copy(zb.at[pl.ds(0, 400)],
                        degp_hbm.at[pl.ds(c * N + 9600, 400)])


_sc_deg = pl.kernel(
    _sc_deg_body,
    out_type=jax.ShapeDtypeStruct((2 * N,), jnp.float32),
    mesh=_mesh,
    scratch_types=[pltpu.VMEM((128,), jnp.float32),
                   pltpu.VMEM((157, 128), jnp.int32),
                   pltpu.VMEM((640,), jnp.float32),
                   pltpu.VMEM_SHARED((N + 16,), jnp.float32),
                   pltpu.SemaphoreType.DMA],
    compiler_params=pltpu.CompilerParams(use_tc_tiling_on_sc=False),
)


QW = 64
KE = 128
EPT = E // NS
NCH = (EPT + KE - 1) // KE
EPAD = NCH * KE - EPT
NBUF = 6
NGRP = NCH // NBUF
NEPI = NCH - NGRP * NBUF
DUMP = N


def _zero_g0(g0):
    def zrow(r, carry):
        for cc in range(QW // 16):
            g0[r, pl.ds(cc * 16, 16)] = jnp.zeros((16,), jnp.float32)
        return carry

    lax.fori_loop(0, KE, zrow, 0)


def _acc_rows(s, fn):
    r0 = s * 640

    @pl.when(s < 15)
    def _():
        for i in range(5):
            fn(r0 + i * KE, KE)

    @pl.when(s == 15)
    def _():
        for i in range(3):
            fn(9600 + i * KE, KE)
        fn(9984, 16)


def _agg_pass(q_hbm, out_hbm, sb2, db2, gbufs, gsems, ssems, acc_sh, s):
    g0 = gbufs[0]
    _zero_g0(g0)
    _acc_rows(s, lambda r, n: pltpu.sync_copy(
        g0.at[pl.ds(0, n)], acc_sh.at[pl.ds(r, n)]))
    plsc.subcore_barrier()

    for b in range(NBUF):
        pltpu.async_copy(q_hbm.at[sb2.at[b]], gbufs[b], gsems[b])

    def group(j, carry):
        descs = []
        for b in range(NBUF):
            k = NBUF * j + b
            pltpu.make_async_copy(q_hbm.at[sb2.at[0]], gbufs[b],
                                  gsems[b]).wait()
            descs.append(pltpu.async_copy(
                gbufs[b], acc_sh.at[db2.at[k]], ssems[b], add=True))
        for b in range(NBUF):
            k = NBUF * j + b
            descs[b].wait()

            @pl.when(k + NBUF < NCH)
            def _(b=b, k=k):
                pltpu.async_copy(q_hbm.at[sb2.at[k + NBUF]], gbufs[b],
                                 gsems[b])
        return carry

    lax.fori_loop(0, NGRP, group, 0)
    edescs = []
    for b in range(NEPI):
        pltpu.make_async_copy(q_hbm.at[sb2.at[0]], gbufs[b], gsems[b]).wait()
        edescs.append(pltpu.async_copy(
            gbufs[b], acc_sh.at[db2.at[NGRP * NBUF + b]], ssems[b], add=True))
    for d in edescs:
        d.wait()
    plsc.subcore_barrier()

    def wb(r, n):
        pltpu.sync_copy(acc_sh.at[pl.ds(r, n)], g0.at[pl.ds(0, n)])
        pltpu.sync_copy(g0.at[pl.ds(0, n)], out_hbm.at[pl.ds(r, n)])

    _acc_rows(s, wb)


def _edge_body(src3d_hbm, dst3d_hbm, q0_hbm, q1_hbm, q2_hbm, q3_hbm,
               a0_hbm, a1_hbm, a2_hbm, a3_hbm,
               sb2, db2, g0, g1, g2, g3, g4, g5, acc_sh,
               gs0, gs1, gs2, gs3, gs4, gs5,
               ss0, ss1, ss2, ss3, ss4, ss5):
    c = lax.axis_index("c")
    s = lax.axis_index("s")
    gbufs = (g0, g1, g2, g3, g4, g5)
    gsems = (gs0, gs1, gs2, gs3, gs4, gs5)
    ssems = (ss0, ss1, ss2, ss3, ss4, ss5)
    pltpu.sync_copy(src3d_hbm.at[s], sb2)
    pltpu.sync_copy(dst3d_hbm.at[s], db2)

    @pl.when(c == 0)
    def _():
        _agg_pass(q0_hbm, a0_hbm, sb2, db2, gbufs, gsems, ssems, acc_sh, s)
        plsc.subcore_barrier()
        _agg_pass(q1_hbm, a1_hbm, sb2, db2, gbufs, gsems, ssems, acc_sh, s)

    @pl.when(c == 1)
    def _():
        _agg_pass(q2_hbm, a2_hbm, sb2, db2, gbufs, gsems, ssems, acc_sh, s)
        plsc.subcore_barrier()
        _agg_pass(q3_hbm, a3_hbm, sb2, db2, gbufs, gsems, ssems, acc_sh, s)


_edge_scratch = [pltpu.VMEM((NCH, KE), jnp.int32),
                 pltpu.VMEM((NCH, KE), jnp.int32)] + \
                [pltpu.VMEM((KE, QW), jnp.float32)] * NBUF + \
                [pltpu.VMEM_SHARED((N + 16, QW), jnp.float32)] + \
                [pltpu.SemaphoreType.DMA] * (2 * NBUF)

_edge_agg = pl.kernel(
    _edge_body,
    out_type=[jax.ShapeDtypeStruct((N, QW), jnp.float32)] * 4,
    mesh=_mesh,
    scratch_types=_edge_scratch,
    compiler_params=pltpu.CompilerParams(use_tc_tiling_on_sc=False),
)


BLK = 1000
NB = N // BLK


def _prescale_body(x_ref, emb_ref, dpT_ref, s_ref, qa_ref, qb_ref):
    vids = lax.broadcasted_iota(jnp.int32, (BLK, 128), 1)
    h = jnp.zeros((BLK, EMB), jnp.float32)
    for f in range(NF):
        oh = (x_ref[:, f:f + 1] == vids).astype(jnp.float32)
        h = h + jnp.dot(oh, emb_ref[f])
    d = dpT_ref[:, 0:1] + dpT_ref[:, 1:2]
    s = lax.rsqrt(1.0 + d)
    s_ref[...] = s
    q = h * s
    qa_ref[...] = q[:, :QW]
    qb_ref[...] = q[:, QW:]


def _tc_prescale(x, emb_pad, dpT):
    return pl.pallas_call(
        _prescale_body,
        grid=(NB,),
        in_specs=[pl.BlockSpec((BLK, NF), lambda i: (i, 0)),
                  pl.BlockSpec((NF, 128, EMB), lambda i: (0, 0, 0)),
                  pl.BlockSpec((BLK, 2), lambda i: (i, 0))],
        out_specs=[pl.BlockSpec((BLK, 1), lambda i: (i, 0)),
                   pl.BlockSpec((BLK, QW), lambda i: (i, 0)),
                   pl.BlockSpec((BLK, QW), lambda i: (i, 0))],
        out_shape=[jax.ShapeDtypeStruct((N, 1), jnp.float32),
                   jax.ShapeDtypeStruct((N, QW), jnp.float32),
                   jax.ShapeDtypeStruct((N, QW), jnp.float32)],
    )(x, emb_pad, dpT)


def _layer0_body(a0, a1, q0, q1, s_ref, w_ref, b_ref, o0, o1, o2, o3):
    s = s_ref[...]
    z = jnp.concatenate([a0[...] + q0[...], a1[...] + q1[...]], axis=1) * s
    h = jnp.maximum(jnp.dot(z, w_ref[...]) + b_ref[...], 0.0)
    qn = h * s
    o0[...] = qn[:, 0 * QW:1 * QW]
    o1[...] = qn[:, 1 * QW:2 * QW]
    o2[...] = qn[:, 2 * QW:3 * QW]
    o3[...] = qn[:, 3 * QW:4 * QW]


def _layer_body(a0, a1, a2, a3, q0, q1, q2, q3, s_ref, w_ref, b_ref,
                o0, o1, o2, o3):
    s = s_ref[...]
    z = jnp.concatenate([a0[...] + q0[...], a1[...] + q1[...],
                         a2[...] + q2[...], a3[...] + q3[...]], axis=1) * s
    h = jnp.maximum(jnp.dot(z, w_ref[...]) + b_ref[...], 0.0)
    qn = h * s
    o0[...] = qn[:, 0 * QW:1 * QW]
    o1[...] = qn[:, 1 * QW:2 * QW]
    o2[...] = qn[:, 2 * QW:3 * QW]
    o3[...] = qn[:, 3 * QW:4 * QW]


_qspec = pl.BlockSpec((BLK, QW), lambda i: (i, 0))


def _tc_layer0(aq, qq, s, wm, bias):
    return pl.pallas_call(
        _layer0_body,
        grid=(NB,),
        in_specs=[_qspec] * 4 + [
            pl.BlockSpec((BLK, 1), lambda i: (i, 0)),
            pl.BlockSpec((EMB, HID), lambda i: (0, 0)),
            pl.BlockSpec((1, HID), lambda i: (0, 0))],
        out_specs=[_qspec] * 4,
        out_shape=[jax.ShapeDtypeStruct((N, QW), jnp.float32)] * 4,
    )(*aq, *qq, s, wm, bias.reshape(1, HID))


def _tc_layer(aq, qq, s, wm, bias):
    return pl.pallas_call(
        _layer_body,
        grid=(NB,),
        in_specs=[_qspec] * 8 + [
            pl.BlockSpec((BLK, 1), lambda i: (i, 0)),
            pl.BlockSpec((HID, HID), lambda i: (0, 0)),
            pl.BlockSpec((1, HID), lambda i: (0, 0))],
        out_specs=[_qspec] * 4,
        out_shape=[jax.ShapeDtypeStruct((N, QW), jnp.float32)] * 4,
    )(*aq, *qq, s, wm, bias.reshape(1, HID))


def _readout_body(q0, q1, q2, q3, s_ref, bidx_ref, linw_ref, linb_ref,
                  out_ref, sums_scr, cnts_scr):
    i = pl.program_id(0)
    s = s_ref[...]
    h3 = jnp.concatenate([q0[...], q1[...], q2[...], q3[...]], axis=1) / s
    gids = lax.broadcasted_iota(jnp.int32, (NG, BLK), 0)
    ohT = (bidx_ref[0] == gids).astype(jnp.float32)
    bs = jnp.dot(ohT, h3)
    bc = jnp.sum(ohT, axis=1, keepdims=True)

    @pl.when(i == 0)
    def _():
        sums_scr[...] = bs
        cnts_scr[...] = bc

    @pl.when(i > 0)
    def _():
        sums_scr[...] += bs
        cnts_scr[...] += bc

    @pl.when(i == NB - 1)
    def _():
        mean = sums_scr[...] / jnp.maximum(cnts_scr[...], 1.0)
        out_ref[...] = jax.nn.sigmoid(jnp.dot(mean, linw_ref[...])
                                      + linb_ref[...])


def _tc_readout(qq, s, bidx_3d, lin_W, lin_b):
    return pl.pallas_call(
        _readout_body,
        grid=(NB,),
        in_specs=[_qspec] * 4 + [
            pl.BlockSpec((BLK, 1), lambda i: (i, 0)),
            pl.BlockSpec((1, 1, BLK), lambda i: (i, 0, 0)),
            pl.BlockSpec((HID, 1), lambda i: (0, 0)),
            pl.BlockSpec((1, 1), lambda i: (0, 0))],
        out_specs=pl.BlockSpec((NG, 1), lambda i: (0, 0)),
        out_shape=jax.ShapeDtypeStruct((NG, 1), jnp.float32),
        scratch_shapes=[pltpu.VMEM((NG, HID), jnp.float32),
                        pltpu.VMEM((NG, 1), jnp.float32)],
    )(*qq, s, bidx_3d, lin_W, lin_b.reshape(1, 1))


def kernel(x, edge_index, batch_idx, atom_emb, W0, b0, W1, b1, W2, b2,
           lin_W, lin_b):
    x = x.astype(jnp.int32)
    src = edge_index[0].astype(jnp.int32)
    dst = edge_index[1].astype(jnp.int32)
    emb_pad = jnp.pad(atom_emb, ((0, 0), (0, 128 - VOCAB), (0, 0)))

    srcr = src.reshape(NS, EPT)
    dstr = dst.reshape(NS, EPT)
    src3d = jnp.concatenate(
        [srcr, jnp.zeros((NS, EPAD), jnp.int32)], axis=1).reshape(NS, NCH, KE)
    dst3d = jnp.concatenate(
        [dstr, jnp.full((NS, EPAD), DUMP, jnp.int32)],
        axis=1).reshape(NS, NCH, KE)

    degp = _sc_deg(dst3d)
    s, q0a, q0b = _tc_prescale(x, emb_pad, jnp.transpose(degp.reshape(2, N)))

    zq = jnp.zeros((N, QW), jnp.float32)
    aq = _edge_agg(src3d, dst3d, q0a, q0b, zq, zq)
    carry = _tc_layer0(aq[:2], (q0a, q0b), s, W0, b0)
    for wm, bias in ((W1, b1), (W2, b2)):
        aq = _edge_agg(src3d, dst3d, *carry)
        carry = _tc_layer(aq, carry, s, wm, bias)
    q3 = carry

    out = _tc_readout(q3, s,
                      batch_idx.astype(jnp.int32).reshape(NB, 1, BLK),
                      lin_W, lin_b)
    return out
